# baseline (speedup 1.0000x reference)
import jax
import jax.numpy as jnp
from jax import lax
from jax.experimental import pallas as pl
from jax.experimental.pallas import tpu as pltpu
from jax.experimental.pallas import tpu_sc as plsc

N_USERS = 2000
N_ITEMS = 8000
NN = N_USERS + N_ITEMS
D = 128
HOPS = 3
NNEG = 4
BATCH = 1024
E = 320000

NC = 2
NS = 16
NW = NC * NS
HALF = NN // NC
ACC_ROWS = 5024
EC = 32

EPB = E // NW
IC = 80
RCAP = 10048
RCH = RCAP // EC
HSTRIDE = NW * RCAP
BLEN = NC * HSTRIDE + EC

XROWS_PER_SUB = 624
XTAIL = NN - NS * XROWS_PER_SUB


def _bucket_body(row_hbm, col_hbm, val_hbm,
                 bw0_out, bval_out, counts_out,
                 irow, icol, ival, sw0_0, sv_0, sw0_1, sv_1,
                 aw0_0, av_0, aw0_1, av_1, cbuf, sm):
    c = lax.axis_index("c")
    s = lax.axis_index("s")
    w = s * NC + c
    ebase = w * EPB
    lane = lax.iota(jnp.int32, 16)
    sw0 = (sw0_0, sw0_1)
    sv = (sv_0, sv_1)
    aw0 = (aw0_0, aw0_1)
    av = (av_0, av_1)
    sm[0] = 0
    sm[1] = 0
    sm[2] = 0
    sm[3] = 0

    def insert(h, w0s, vs):
        fc = sm[2 * h]
        a = aw0[h][pl.ds(0, 16)]
        b = av[h][pl.ds(0, 16)]
        aw0[h][pl.ds(0, 16)] = jnp.where(lane == fc, w0s, a)
        av[h][pl.ds(0, 16)] = jnp.where(lane == fc, vs, b)

        @pl.when(fc == 15)
        def _():
            off = sm[2 * h + 1]
            sw0[h][pl.ds(off, 16)] = aw0[h][pl.ds(0, 16)]
            sv[h][pl.ds(off, 16)] = av[h][pl.ds(0, 16)]
            sm[2 * h + 1] = off + 16

        sm[2 * h] = (fc + 1) & 15

    def chunk(j, carry):
        off = ebase + j * IC
        pltpu.sync_copy(row_hbm.at[pl.ds(off, IC)], irow)
        pltpu.sync_copy(col_hbm.at[pl.ds(off, IC)], icol)
        pltpu.sync_copy(val_hbm.at[pl.ds(off, IC)], ival)
        for g in range(IC // 16):
            r = irow[pl.ds(g * 16, 16)]
            cv = icol[pl.ds(g * 16, 16)]
            vv = ival[pl.ds(g * 16, 16)]
            lr = jnp.where(r >= HALF, r - HALF, r)
            w0v = lax.shift_left(lr, 14) + cv
            for l in range(16):
                w0s = w0v[l]
                vs = vv[l]
                local0 = r[l] < HALF

                @pl.when(local0)
                def _(w0s=w0s, vs=vs):
                    insert(0, w0s, vs)

                @pl.when(jnp.logical_not(local0))
                def _(w0s=w0s, vs=vs):
                    insert(1, w0s, vs)
        return carry

    lax.fori_loop(0, EPB // IC, chunk, 0)

    trash_w0 = jnp.broadcast_to((HALF + (w % NS)) << 14, (16,)).astype(jnp.int32)
    zero_f = jnp.zeros((16,), jnp.float32)
    for h in range(NC):
        fc = sm[2 * h]
        off = sm[2 * h + 1]
        a = aw0[h][pl.ds(0, 16)]
        b = av[h][pl.ds(0, 16)]
        sw0[h][pl.ds(off, 16)] = jnp.where(lane < fc, a, trash_w0)
        sv[h][pl.ds(off, 16)] = jnp.where(lane < fc, b, zero_f)
        sw0[h][pl.ds(off + 16, 16)] = trash_w0
        sv[h][pl.ds(off + 16, 16)] = zero_f
        sw0[h][pl.ds(off + 32, 16)] = trash_w0
        sv[h][pl.ds(off + 32, 16)] = zero_f
        base = h * HSTRIDE + w * RCAP
        pltpu.sync_copy(sw0[h], bw0_out.at[pl.ds(base, RCAP)])
        pltpu.sync_copy(sv[h], bval_out.at[pl.ds(base, RCAP)])
        nch = (off + fc + (EC - 1)) // EC
        cbuf[pl.ds(0, 16)] = jnp.where(lane == 0, nch, 0)
        pltpu.sync_copy(cbuf, counts_out.at[pl.ds((h * NW + w) * 16, 16)])


_bucket = pl.kernel(
    _bucket_body,
    out_type=(jax.ShapeDtypeStruct((BLEN,), jnp.int32),
              jax.ShapeDtypeStruct((BLEN,), jnp.float32),
              jax.ShapeDtypeStruct((NC * NW * 16,), jnp.int32)),
    mesh=plsc.VectorSubcoreMesh(core_axis_name="c", subcore_axis_name="s"),
    scratch_types=[
        pltpu.VMEM((IC,), jnp.int32),
        pltpu.VMEM((IC,), jnp.int32),
        pltpu.VMEM((IC,), jnp.float32),
        pltpu.VMEM((RCAP,), jnp.int32),
        pltpu.VMEM((RCAP,), jnp.float32),
        pltpu.VMEM((RCAP,), jnp.int32),
        pltpu.VMEM((RCAP,), jnp.float32),
        pltpu.VMEM((16,), jnp.int32),
        pltpu.VMEM((16,), jnp.float32),
        pltpu.VMEM((16,), jnp.int32),
        pltpu.VMEM((16,), jnp.float32),
        pltpu.VMEM((16,), jnp.int32),
        pltpu.SMEM((8,), jnp.int32),
    ],
)


def _hop_body(x_hbm, bw0, bval, counts, zero_hbm, out_hbm,
              x_sh, acc_sh, cbuf, tridx,
              w0A, lrowA, colA, valA, rowsA,
              w0B, lrowB, colB, valB, rowsB,
              semMA, semMB, semGA, semGB, semSA, semSB):
    c = lax.axis_index("c")
    s = lax.axis_index("s")
    lo = c * HALF
    trash = HALF + s

    pltpu.sync_copy(x_hbm.at[pl.ds(s * XROWS_PER_SUB, XROWS_PER_SUB)],
                    x_sh.at[pl.ds(s * XROWS_PER_SUB, XROWS_PER_SUB)])

    @pl.when(s == 0)
    def _():
        pltpu.sync_copy(x_hbm.at[pl.ds(NS * XROWS_PER_SUB, XTAIL)],
                        x_sh.at[pl.ds(NS * XROWS_PER_SUB, XTAIL)])

    @pl.when(s == 1)
    def _():
        pltpu.sync_copy(zero_hbm, acc_sh)

    for t in range(EC // 16):
        tridx[pl.ds(t * 16, 16)] = jnp.broadcast_to(trash, (16,)).astype(jnp.int32)
    plsc.subcore_barrier()

    A = (w0A, lrowA, colA, valA, rowsA, semMA, semGA, semSA)
    Bb = (w0B, lrowB, colB, valB, rowsB, semMB, semGB, semSB)

    def meta_issue(rbase, j, bufs):
        w0b, valb, semM = bufs[0], bufs[3], bufs[5]
        off = rbase + j * EC
        pltpu.async_copy(bw0.at[pl.ds(off, EC)], w0b, semM)
        pltpu.async_copy(bval.at[pl.ds(off, EC)], valb, semM)

    def meta_drain(rbase, bufs):
        w0b, valb, semM = bufs[0], bufs[3], bufs[5]
        pltpu.make_async_copy(bw0.at[pl.ds(rbase, EC)], w0b, semM).wait()
        pltpu.make_async_copy(bval.at[pl.ds(rbase, EC)], valb, semM).wait()

    def start_phase(rbase, bufs):
        w0b, lrowb, colb, rowsb = bufs[0], bufs[1], bufs[2], bufs[4]
        semG, semS = bufs[6], bufs[7]
        meta_drain(rbase, bufs)
        for g in range(EC // 16):
            wv = w0b[pl.ds(g * 16, 16)]
            lrowb[pl.ds(g * 16, 16)] = lax.shift_right_logical(wv, 14)
            colb[pl.ds(g * 16, 16)] = wv & 16383
        pltpu.make_async_copy(rowsb, acc_sh.at[lrowb], semS).wait()
        pltpu.async_copy(x_sh.at[colb], rowsb, semG)

    def finish_phase(rbase, bufs, next_meta_j):
        lrowb, colb, valb, rowsb = bufs[1], bufs[2], bufs[3], bufs[4]
        semG, semS = bufs[6], bufs[7]
        pltpu.make_async_copy(x_sh.at[colb], rowsb, semG).wait()

        def scale16(g, _):
            vv = valb[pl.ds(g * 16, 16)]
            for l in range(16):
                e = g * 16 + l
                v = vv[l]
                for q in range(D // 16):
                    rowsb[e, pl.ds(q * 16, 16)] = rowsb[e, pl.ds(q * 16, 16)] * v
            return 0

        lax.fori_loop(0, EC // 16, scale16, 0)
        pltpu.async_copy(rowsb, acc_sh.at[lrowb], semS)
        if next_meta_j is not None:
            meta_issue(rbase, next_meta_j, bufs)

    def do_region(ridx, nch):
        rbase = c * HSTRIDE + ridx * RCAP
        pltpu.async_copy(rowsA, acc_sh.at[tridx], semSA)
        pltpu.async_copy(rowsB, acc_sh.at[tridx], semSB)
        meta_issue(rbase, 0, A)
        meta_issue(rbase, 1, Bb)

        def pair(k, carry):
            start_phase(rbase, A)
            start_phase(rbase, Bb)
            finish_phase(rbase, A, 2 * k + 2)
            finish_phase(rbase, Bb, 2 * k + 3)
            return carry

        lax.fori_loop(0, nch // 2, pair, 0)
        odd = nch & 1

        @pl.when(odd == 1)
        def _():
            start_phase(rbase, A)
            finish_phase(rbase, A, None)

        pltpu.make_async_copy(rowsA, acc_sh.at[lrowA], semSA).wait()
        pltpu.make_async_copy(rowsB, acc_sh.at[lrowB], semSB).wait()
        meta_drain(rbase, Bb)

        @pl.when(odd == 0)
        def _():
            meta_drain(rbase, A)

    pltpu.sync_copy(counts.at[pl.ds((c * NW + 2 * s) * 16, 32)], cbuf)
    n0 = cbuf[pl.ds(0, 16)][0]
    n1 = cbuf[pl.ds(16, 16)][0]
    do_region(2 * s, n0)
    do_region(2 * s + 1, n1)
    plsc.subcore_barrier()

    @pl.when(s == 0)
    def _():
        pltpu.sync_copy(acc_sh.at[pl.ds(0, HALF)], out_hbm.at[pl.ds(lo, HALF)])


_hop = pl.kernel(
    _hop_body,
    out_type=jax.ShapeDtypeStruct((NN, D), jnp.float32),
    mesh=plsc.VectorSubcoreMesh(core_axis_name="c", subcore_axis_name="s"),
    scratch_types=[
        pltpu.VMEM_SHARED((NN, D), jnp.float32),
        pltpu.VMEM_SHARED((ACC_ROWS, D), jnp.float32),
        pltpu.VMEM((32,), jnp.int32),
        pltpu.VMEM((EC,), jnp.int32),
        pltpu.VMEM((EC,), jnp.int32),
        pltpu.VMEM((EC,), jnp.int32),
        pltpu.VMEM((EC,), jnp.int32),
        pltpu.VMEM((EC,), jnp.float32),
        pltpu.VMEM((EC, D), jnp.float32),
        pltpu.VMEM((EC,), jnp.int32),
        pltpu.VMEM((EC,), jnp.int32),
        pltpu.VMEM((EC,), jnp.int32),
        pltpu.VMEM((EC,), jnp.float32),
        pltpu.VMEM((EC, D), jnp.float32),
        pltpu.SemaphoreType.DMA,
        pltpu.SemaphoreType.DMA,
        pltpu.SemaphoreType.DMA,
        pltpu.SemaphoreType.DMA,
        pltpu.SemaphoreType.DMA,
        pltpu.SemaphoreType.DMA,
    ],
)

BPW = BATCH // NW


def _loss_body(x0, x1, x2, x3, u_hbm, i_hbm, n_hbm, pos_out, neg_out,
               idxv, tmp, usum, isum, ng0, ng1, ng2, ng3, pos_sm, neg_sm, sem):
    c = lax.axis_index("c")
    s = lax.axis_index("s")
    w = s * NC + c
    b0 = w * BPW
    tables = (x0, x1, x2, x3)
    ngs = (ng0, ng1, ng2, ng3)

    def gather_sum(idx_hbm, off, dst):
        pltpu.sync_copy(idx_hbm.at[pl.ds(off, BPW)], idxv)
        pltpu.async_copy(tables[0].at[idxv], dst, sem).wait()
        for t in range(1, 4):
            pltpu.async_copy(tables[t].at[idxv], tmp, sem).wait()

            def addloop(i, _):
                for q in range(D // 16):
                    dst[i, pl.ds(q * 16, 16)] = (dst[i, pl.ds(q * 16, 16)]
                                                 + tmp[i, pl.ds(q * 16, 16)])
                return 0

            lax.fori_loop(0, BPW, addloop, 0)

    gather_sum(u_hbm, b0, usum)
    gather_sum(i_hbm, b0, isum)
    for n in range(NNEG):
        gather_sum(n_hbm, n * BATCH + b0, ngs[n])

    def dots(b, _):
        pacc = jnp.zeros((16,), jnp.float32)
        for q in range(D // 16):
            pacc = pacc + (usum[b, pl.ds(q * 16, 16)]
                           * isum[b, pl.ds(q * 16, 16)])
        pos_sm[b, pl.ds(0, 16)] = pacc
        for n in range(NNEG):
            nacc = jnp.zeros((16,), jnp.float32)
            for q in range(D // 16):
                nacc = nacc + (usum[b, pl.ds(q * 16, 16)]
                               * ngs[n][b, pl.ds(q * 16, 16)])
            neg_sm[n * BPW + b, pl.ds(0, 16)] = nacc
        return 0

    lax.fori_loop(0, BPW, dots, 0)
    pltpu.sync_copy(pos_sm, pos_out.at[pl.ds(b0, BPW)])
    for n in range(NNEG):
        pltpu.sync_copy(neg_sm.at[pl.ds(n * BPW, BPW)],
                        neg_out.at[pl.ds(n * BATCH + b0, BPW)])


_loss = pl.kernel(
    _loss_body,
    out_type=(jax.ShapeDtypeStruct((BATCH, 16), jnp.float32),
              jax.ShapeDtypeStruct((NNEG * BATCH, 16), jnp.float32)),
    mesh=plsc.VectorSubcoreMesh(core_axis_name="c", subcore_axis_name="s"),
    scratch_types=[
        pltpu.VMEM((BPW,), jnp.int32),
        pltpu.VMEM((BPW, D), jnp.float32),
        pltpu.VMEM((BPW, D), jnp.float32),
        pltpu.VMEM((BPW, D), jnp.float32),
        pltpu.VMEM((BPW, D), jnp.float32),
        pltpu.VMEM((BPW, D), jnp.float32),
        pltpu.VMEM((BPW, D), jnp.float32),
        pltpu.VMEM((BPW, D), jnp.float32),
        pltpu.VMEM((BPW, 16), jnp.float32),
        pltpu.VMEM((NNEG * BPW, 16), jnp.float32),
        pltpu.SemaphoreType.DMA,
    ],
)


def _nce_body(p_ref, n_ref, o_ref):
    p = jnp.sum(p_ref[...], axis=-1) * (1.0 / 16.0)
    nk = jnp.sum(n_ref[...], axis=-1) * (1.0 / 16.0)
    ne = jnp.sum(jnp.exp(nk), axis=0)
    loss = jnp.mean(jnp.log(jnp.exp(p) + ne) - p)
    o_ref[...] = jnp.reshape(loss, (1, 1))


_nce = pl.pallas_call(
    _nce_body,
    out_shape=jax.ShapeDtypeStruct((1, 1), jnp.float32),
)


def kernel(edge_vals, user_emb, item_emb, users, items, negatives, edge_index):
    all_emb = jnp.concatenate([user_emb, item_emb], axis=0).astype(jnp.float32)
    row = edge_index[0].astype(jnp.int32)
    col = edge_index[1].astype(jnp.int32)
    ev = edge_vals.astype(jnp.float32)
    zero_acc = jnp.zeros((ACC_ROWS, D), jnp.float32)

    bw0, bval, counts = _bucket(row, col, ev)

    x0 = all_emb
    x1 = _hop(x0, bw0, bval, counts, zero_acc)
    x2 = _hop(x1, bw0, bval, counts, zero_acc)
    x3 = _hop(x2, bw0, bval, counts, zero_acc)

    u = users.astype(jnp.int32)
    it = items.astype(jnp.int32) + N_USERS
    ng = negatives.astype(jnp.int32) + N_USERS
    pos, negk = _loss(x0, x1, x2, x3, u, it, ng)
    out = _nce(pos, negk.reshape(NNEG, BATCH, 16))
    return out[0, 0]


# speedup vs baseline: 1.5775x; 1.5775x over previous
import jax
import jax.numpy as jnp
from jax import lax
from jax.experimental import pallas as pl
from jax.experimental.pallas import tpu as pltpu
from jax.experimental.pallas import tpu_sc as plsc

N_USERS = 2000
N_ITEMS = 8000
NN = N_USERS + N_ITEMS
D = 128
HOPS = 3
NNEG = 4
BATCH = 1024
E = 320000

NC = 2
NS = 16
NW = NC * NS
HALF = NN // NC
ACC_ROWS = 5024
EC = 32

EPB = E // NW
IC = 80
RCAP = 10048
RCH = RCAP // EC
HSTRIDE = NW * RCAP
BLEN = NC * HSTRIDE + EC

XROWS_PER_SUB = 624
XTAIL = NN - NS * XROWS_PER_SUB


def _bucket_body(row_hbm, col_hbm, val_hbm,
                 bw0_out, bval_out, counts_out,
                 irow, icol, ival, sw0_0, sv_0, sw0_1, sv_1, cbuf):
    c = lax.axis_index("c")
    s = lax.axis_index("s")
    w = s * NC + c
    ebase = w * EPB
    lane = lax.iota(jnp.int32, 16)
    sw0 = (sw0_0, sw0_1)
    sv = (sv_0, sv_1)
    zi = jnp.zeros((16,), jnp.int32)
    zf = jnp.zeros((16,), jnp.float32)

    def chunk(j, carry):
        (fc0, off0, w0lo, w0hi, v0lo, v0hi,
         fc1, off1, w1lo, w1hi, v1lo, v1hi) = carry
        off = ebase + j * IC
        pltpu.sync_copy(row_hbm.at[pl.ds(off, IC)], irow)
        pltpu.sync_copy(col_hbm.at[pl.ds(off, IC)], icol)
        pltpu.sync_copy(val_hbm.at[pl.ds(off, IC)], ival)
        for g in range(IC // 16):
            r = irow[pl.ds(g * 16, 16)]
            cv = icol[pl.ds(g * 16, 16)]
            vv = ival[pl.ds(g * 16, 16)]
            lr = jnp.where(r >= HALF, r - HALF, r)
            w0v = lax.shift_left(lr, 14) + cv
            for l in range(16):
                w0s = w0v[l]
                vs = vv[l]
                loc = r[l] < HALF
                t0 = jnp.where(loc, fc0, -1)
                t1 = jnp.where(loc, -1, fc1)
                m0lo = lane == t0
                m0hi = lane == (t0 - 16)
                m1lo = lane == t1
                m1hi = lane == (t1 - 16)
                w0lo = jnp.where(m0lo, w0s, w0lo)
                w0hi = jnp.where(m0hi, w0s, w0hi)
                v0lo = jnp.where(m0lo, vs, v0lo)
                v0hi = jnp.where(m0hi, vs, v0hi)
                w1lo = jnp.where(m1lo, w0s, w1lo)
                w1hi = jnp.where(m1hi, w0s, w1hi)
                v1lo = jnp.where(m1lo, vs, v1lo)
                v1hi = jnp.where(m1hi, vs, v1hi)
                inc = jnp.where(loc, 1, 0)
                fc0 = fc0 + inc
                fc1 = fc1 + (1 - inc)
            fl0 = fc0 >= 16

            @pl.when(fl0)
            def _(w0lo=w0lo, v0lo=v0lo, off0=off0):
                sw0[0][pl.ds(off0, 16)] = w0lo
                sv[0][pl.ds(off0, 16)] = v0lo

            w0lo = jnp.where(fl0, w0hi, w0lo)
            v0lo = jnp.where(fl0, v0hi, v0lo)
            fc0 = jnp.where(fl0, fc0 - 16, fc0)
            off0 = jnp.where(fl0, off0 + 16, off0)
            fl1 = fc1 >= 16

            @pl.when(fl1)
            def _(w1lo=w1lo, v1lo=v1lo, off1=off1):
                sw0[1][pl.ds(off1, 16)] = w1lo
                sv[1][pl.ds(off1, 16)] = v1lo

            w1lo = jnp.where(fl1, w1hi, w1lo)
            v1lo = jnp.where(fl1, v1hi, v1lo)
            fc1 = jnp.where(fl1, fc1 - 16, fc1)
            off1 = jnp.where(fl1, off1 + 16, off1)
        return (fc0, off0, w0lo, w0hi, v0lo, v0hi,
                fc1, off1, w1lo, w1hi, v1lo, v1hi)

    init = (jnp.int32(0), jnp.int32(0), zi, zi, zf, zf,
            jnp.int32(0), jnp.int32(0), zi, zi, zf, zf)
    (fc0, off0, w0lo, _, v0lo, _,
     fc1, off1, w1lo, _, v1lo, _) = lax.fori_loop(0, EPB // IC, chunk, init)

    trash_w0 = jnp.broadcast_to((HALF + (w % NS)) << 14, (16,)).astype(jnp.int32)
    for h, fc, off, wlo, vlo in ((0, fc0, off0, w0lo, v0lo),
                                 (1, fc1, off1, w1lo, v1lo)):
        sw0[h][pl.ds(off, 16)] = jnp.where(lane < fc, wlo, trash_w0)
        sv[h][pl.ds(off, 16)] = jnp.where(lane < fc, vlo, zf)
        sw0[h][pl.ds(off + 16, 16)] = trash_w0
        sv[h][pl.ds(off + 16, 16)] = zf
        sw0[h][pl.ds(off + 32, 16)] = trash_w0
        sv[h][pl.ds(off + 32, 16)] = zf
        base = h * HSTRIDE + w * RCAP
        pltpu.sync_copy(sw0[h], bw0_out.at[pl.ds(base, RCAP)])
        pltpu.sync_copy(sv[h], bval_out.at[pl.ds(base, RCAP)])
        nch = (off + fc + (EC - 1)) // EC
        cbuf[pl.ds(0, 16)] = jnp.where(lane == 0, nch, 0)
        pltpu.sync_copy(cbuf, counts_out.at[pl.ds((h * NW + w) * 16, 16)])


_bucket = pl.kernel(
    _bucket_body,
    out_type=(jax.ShapeDtypeStruct((BLEN,), jnp.int32),
              jax.ShapeDtypeStruct((BLEN,), jnp.float32),
              jax.ShapeDtypeStruct((NC * NW * 16,), jnp.int32)),
    mesh=plsc.VectorSubcoreMesh(core_axis_name="c", subcore_axis_name="s"),
    scratch_types=[
        pltpu.VMEM((IC,), jnp.int32),
        pltpu.VMEM((IC,), jnp.int32),
        pltpu.VMEM((IC,), jnp.float32),
        pltpu.VMEM((RCAP,), jnp.int32),
        pltpu.VMEM((RCAP,), jnp.float32),
        pltpu.VMEM((RCAP,), jnp.int32),
        pltpu.VMEM((RCAP,), jnp.float32),
        pltpu.VMEM((16,), jnp.int32),
    ],
)


def _hop_body(x_hbm, bw0, bval, counts, zero_hbm, out_hbm,
              x_sh, acc_sh, cbuf, tridx,
              w0A, lrowA, colA, valA, rowsA,
              w0B, lrowB, colB, valB, rowsB,
              semMA, semMB, semGA, semGB, semSA, semSB):
    c = lax.axis_index("c")
    s = lax.axis_index("s")
    lo = c * HALF
    trash = HALF + s

    pltpu.sync_copy(x_hbm.at[pl.ds(s * XROWS_PER_SUB, XROWS_PER_SUB)],
                    x_sh.at[pl.ds(s * XROWS_PER_SUB, XROWS_PER_SUB)])

    @pl.when(s == 0)
    def _():
        pltpu.sync_copy(x_hbm.at[pl.ds(NS * XROWS_PER_SUB, XTAIL)],
                        x_sh.at[pl.ds(NS * XROWS_PER_SUB, XTAIL)])

    @pl.when(s == 1)
    def _():
        pltpu.sync_copy(zero_hbm, acc_sh)

    for t in range(EC // 16):
        tridx[pl.ds(t * 16, 16)] = jnp.broadcast_to(trash, (16,)).astype(jnp.int32)
    plsc.subcore_barrier()

    A = (w0A, lrowA, colA, valA, rowsA, semMA, semGA, semSA)
    Bb = (w0B, lrowB, colB, valB, rowsB, semMB, semGB, semSB)

    def meta_issue(rbase, j, bufs):
        w0b, valb, semM = bufs[0], bufs[3], bufs[5]
        off = rbase + j * EC
        pltpu.async_copy(bw0.at[pl.ds(off, EC)], w0b, semM)
        pltpu.async_copy(bval.at[pl.ds(off, EC)], valb, semM)

    def meta_drain(rbase, bufs):
        w0b, valb, semM = bufs[0], bufs[3], bufs[5]
        pltpu.make_async_copy(bw0.at[pl.ds(rbase, EC)], w0b, semM).wait()
        pltpu.make_async_copy(bval.at[pl.ds(rbase, EC)], valb, semM).wait()

    def start_phase(rbase, bufs):
        w0b, lrowb, colb, rowsb = bufs[0], bufs[1], bufs[2], bufs[4]
        semG, semS = bufs[6], bufs[7]
        meta_drain(rbase, bufs)
        for g in range(EC // 16):
            wv = w0b[pl.ds(g * 16, 16)]
            lrowb[pl.ds(g * 16, 16)] = lax.shift_right_logical(wv, 14)
            colb[pl.ds(g * 16, 16)] = wv & 16383
        pltpu.make_async_copy(rowsb, acc_sh.at[lrowb], semS).wait()
        pltpu.async_copy(x_sh.at[colb], rowsb, semG)

    def finish_phase(rbase, bufs, next_meta_j):
        lrowb, colb, valb, rowsb = bufs[1], bufs[2], bufs[3], bufs[4]
        semG, semS = bufs[6], bufs[7]
        pltpu.make_async_copy(x_sh.at[colb], rowsb, semG).wait()

        def scale16(g, _):
            vv = valb[pl.ds(g * 16, 16)]
            for l in range(16):
                e = g * 16 + l
                v = vv[l]
                for q in range(D // 16):
                    rowsb[e, pl.ds(q * 16, 16)] = rowsb[e, pl.ds(q * 16, 16)] * v
            return 0

        lax.fori_loop(0, EC // 16, scale16, 0)
        pltpu.async_copy(rowsb, acc_sh.at[lrowb], semS)
        if next_meta_j is not None:
            meta_issue(rbase, next_meta_j, bufs)

    def do_region(ridx, nch):
        rbase = c * HSTRIDE + ridx * RCAP
        pltpu.async_copy(rowsA, acc_sh.at[tridx], semSA)
        pltpu.async_copy(rowsB, acc_sh.at[tridx], semSB)
        meta_issue(rbase, 0, A)
        meta_issue(rbase, 1, Bb)

        def pair(k, carry):
            start_phase(rbase, A)
            start_phase(rbase, Bb)
            finish_phase(rbase, A, 2 * k + 2)
            finish_phase(rbase, Bb, 2 * k + 3)
            return carry

        lax.fori_loop(0, nch // 2, pair, 0)
        odd = nch & 1

        @pl.when(odd == 1)
        def _():
            start_phase(rbase, A)
            finish_phase(rbase, A, None)

        pltpu.make_async_copy(rowsA, acc_sh.at[lrowA], semSA).wait()
        pltpu.make_async_copy(rowsB, acc_sh.at[lrowB], semSB).wait()
        meta_drain(rbase, Bb)

        @pl.when(odd == 0)
        def _():
            meta_drain(rbase, A)

    pltpu.sync_copy(counts.at[pl.ds((c * NW + 2 * s) * 16, 32)], cbuf)
    n0 = cbuf[pl.ds(0, 16)][0]
    n1 = cbuf[pl.ds(16, 16)][0]
    do_region(2 * s, n0)
    do_region(2 * s + 1, n1)
    plsc.subcore_barrier()

    @pl.when(s == 0)
    def _():
        pltpu.sync_copy(acc_sh.at[pl.ds(0, HALF)], out_hbm.at[pl.ds(lo, HALF)])


_hop = pl.kernel(
    _hop_body,
    out_type=jax.ShapeDtypeStruct((NN, D), jnp.float32),
    mesh=plsc.VectorSubcoreMesh(core_axis_name="c", subcore_axis_name="s"),
    scratch_types=[
        pltpu.VMEM_SHARED((NN, D), jnp.float32),
        pltpu.VMEM_SHARED((ACC_ROWS, D), jnp.float32),
        pltpu.VMEM((32,), jnp.int32),
        pltpu.VMEM((EC,), jnp.int32),
        pltpu.VMEM((EC,), jnp.int32),
        pltpu.VMEM((EC,), jnp.int32),
        pltpu.VMEM((EC,), jnp.int32),
        pltpu.VMEM((EC,), jnp.float32),
        pltpu.VMEM((EC, D), jnp.float32),
        pltpu.VMEM((EC,), jnp.int32),
        pltpu.VMEM((EC,), jnp.int32),
        pltpu.VMEM((EC,), jnp.int32),
        pltpu.VMEM((EC,), jnp.float32),
        pltpu.VMEM((EC, D), jnp.float32),
        pltpu.SemaphoreType.DMA,
        pltpu.SemaphoreType.DMA,
        pltpu.SemaphoreType.DMA,
        pltpu.SemaphoreType.DMA,
        pltpu.SemaphoreType.DMA,
        pltpu.SemaphoreType.DMA,
    ],
)

BPW = BATCH // NW


def _loss_body(x0, x1, x2, x3, u_hbm, i_hbm, n_hbm, pos_out, neg_out,
               idxv, tmp, usum, isum, ng0, ng1, ng2, ng3, pos_sm, neg_sm, sem):
    c = lax.axis_index("c")
    s = lax.axis_index("s")
    w = s * NC + c
    b0 = w * BPW
    tables = (x0, x1, x2, x3)
    ngs = (ng0, ng1, ng2, ng3)

    def gather_sum(idx_hbm, off, dst):
        pltpu.sync_copy(idx_hbm.at[pl.ds(off, BPW)], idxv)
        pltpu.async_copy(tables[0].at[idxv], dst, sem).wait()
        for t in range(1, 4):
            pltpu.async_copy(tables[t].at[idxv], tmp, sem).wait()

            def addloop(i, _):
                for q in range(D // 16):
                    dst[i, pl.ds(q * 16, 16)] = (dst[i, pl.ds(q * 16, 16)]
                                                 + tmp[i, pl.ds(q * 16, 16)])
                return 0

            lax.fori_loop(0, BPW, addloop, 0)

    gather_sum(u_hbm, b0, usum)
    gather_sum(i_hbm, b0, isum)
    for n in range(NNEG):
        gather_sum(n_hbm, n * BATCH + b0, ngs[n])

    def dots(b, _):
        pacc = jnp.zeros((16,), jnp.float32)
        for q in range(D // 16):
            pacc = pacc + (usum[b, pl.ds(q * 16, 16)]
                           * isum[b, pl.ds(q * 16, 16)])
        pos_sm[b, pl.ds(0, 16)] = pacc
        for n in range(NNEG):
            nacc = jnp.zeros((16,), jnp.float32)
            for q in range(D // 16):
                nacc = nacc + (usum[b, pl.ds(q * 16, 16)]
                               * ngs[n][b, pl.ds(q * 16, 16)])
            neg_sm[n * BPW + b, pl.ds(0, 16)] = nacc
        return 0

    lax.fori_loop(0, BPW, dots, 0)
    pltpu.sync_copy(pos_sm, pos_out.at[pl.ds(b0, BPW)])
    for n in range(NNEG):
        pltpu.sync_copy(neg_sm.at[pl.ds(n * BPW, BPW)],
                        neg_out.at[pl.ds(n * BATCH + b0, BPW)])


_loss = pl.kernel(
    _loss_body,
    out_type=(jax.ShapeDtypeStruct((BATCH, 16), jnp.float32),
              jax.ShapeDtypeStruct((NNEG * BATCH, 16), jnp.float32)),
    mesh=plsc.VectorSubcoreMesh(core_axis_name="c", subcore_axis_name="s"),
    scratch_types=[
        pltpu.VMEM((BPW,), jnp.int32),
        pltpu.VMEM((BPW, D), jnp.float32),
        pltpu.VMEM((BPW, D), jnp.float32),
        pltpu.VMEM((BPW, D), jnp.float32),
        pltpu.VMEM((BPW, D), jnp.float32),
        pltpu.VMEM((BPW, D), jnp.float32),
        pltpu.VMEM((BPW, D), jnp.float32),
        pltpu.VMEM((BPW, D), jnp.float32),
        pltpu.VMEM((BPW, 16), jnp.float32),
        pltpu.VMEM((NNEG * BPW, 16), jnp.float32),
        pltpu.SemaphoreType.DMA,
    ],
)


def _nce_body(p_ref, n_ref, o_ref):
    p = jnp.sum(p_ref[...], axis=-1) * (1.0 / 16.0)
    nk = jnp.sum(n_ref[...], axis=-1) * (1.0 / 16.0)
    ne = jnp.sum(jnp.exp(nk), axis=0)
    loss = jnp.mean(jnp.log(jnp.exp(p) + ne) - p)
    o_ref[...] = jnp.reshape(loss, (1, 1))


_nce = pl.pallas_call(
    _nce_body,
    out_shape=jax.ShapeDtypeStruct((1, 1), jnp.float32),
)


def kernel(edge_vals, user_emb, item_emb, users, items, negatives, edge_index):
    all_emb = jnp.concatenate([user_emb, item_emb], axis=0).astype(jnp.float32)
    row = edge_index[0].astype(jnp.int32)
    col = edge_index[1].astype(jnp.int32)
    ev = edge_vals.astype(jnp.float32)
    zero_acc = jnp.zeros((ACC_ROWS, D), jnp.float32)

    bw0, bval, counts = _bucket(row, col, ev)

    x0 = all_emb
    x1 = _hop(x0, bw0, bval, counts, zero_acc)
    x2 = _hop(x1, bw0, bval, counts, zero_acc)
    x3 = _hop(x2, bw0, bval, counts, zero_acc)

    u = users.astype(jnp.int32)
    it = items.astype(jnp.int32) + N_USERS
    ng = negatives.astype(jnp.int32) + N_USERS
    pos, negk = _loss(x0, x1, x2, x3, u, it, ng)
    out = _nce(pos, negk.reshape(NNEG, BATCH, 16))
    return out[0, 0]


# speedup vs baseline: 1.9259x; 1.2208x over previous
import jax
import jax.numpy as jnp
from jax import lax
from jax.experimental import pallas as pl
from jax.experimental.pallas import tpu as pltpu
from jax.experimental.pallas import tpu_sc as plsc

N_USERS = 2000
N_ITEMS = 8000
NN = N_USERS + N_ITEMS
D = 128
HOPS = 3
NNEG = 4
BATCH = 1024
E = 320000

NC = 2
NS = 16
NW = NC * NS
HALF = NN // NC
ACC_ROWS = 5024
EC = 32

EPB = E // NW
IC = 80
RCAP = 10048
RCH = RCAP // EC
HSTRIDE = NW * RCAP
BLEN = NC * HSTRIDE + EC

XROWS_PER_SUB = 624
XTAIL = NN - NS * XROWS_PER_SUB


def _bucket_body(row_hbm, col_hbm, val_hbm,
                 bw0_out, bval_out, counts_out,
                 irow, icol, ival, irow2, icol2, ival2,
                 sw0_0, sv_0, sw0_1, sv_1, cbuf, semA, semB):
    c = lax.axis_index("c")
    s = lax.axis_index("s")
    w = s * NC + c
    ebase = w * EPB
    lane = lax.iota(jnp.int32, 16)
    sw0 = (sw0_0, sw0_1)
    sv = (sv_0, sv_1)
    zi = jnp.zeros((16,), jnp.int32)
    zf = jnp.zeros((16,), jnp.float32)

    def meta_issue(j, bufs):
        irowb, icolb, ivalb, semM = bufs
        off = ebase + j * IC
        pltpu.async_copy(row_hbm.at[pl.ds(off, IC)], irowb, semM)
        pltpu.async_copy(col_hbm.at[pl.ds(off, IC)], icolb, semM)
        pltpu.async_copy(val_hbm.at[pl.ds(off, IC)], ivalb, semM)

    def meta_drain(bufs):
        irowb, icolb, ivalb, semM = bufs
        pltpu.make_async_copy(row_hbm.at[pl.ds(0, IC)], irowb, semM).wait()
        pltpu.make_async_copy(col_hbm.at[pl.ds(0, IC)], icolb, semM).wait()
        pltpu.make_async_copy(val_hbm.at[pl.ds(0, IC)], ivalb, semM).wait()

    def process(carry, bufs, next_j):
        (fc0, off0, w0lo, w0hi, v0lo, v0hi,
         fc1, off1, w1lo, w1hi, v1lo, v1hi) = carry
        irowb, icolb, ivalb = bufs[0], bufs[1], bufs[2]
        meta_drain(bufs)
        for g in range(IC // 16):
            r = irowb[pl.ds(g * 16, 16)]
            cv = icolb[pl.ds(g * 16, 16)]
            vv = ivalb[pl.ds(g * 16, 16)]
            lr = jnp.where(r >= HALF, r - HALF, r)
            w0v = lax.shift_left(lr, 14) + cv
            for l in range(16):
                w0s = w0v[l]
                vs = vv[l]
                loc = r[l] < HALF
                t0 = jnp.where(loc, fc0, -1)
                t1 = jnp.where(loc, -1, fc1)
                m0lo = lane == t0
                m0hi = lane == (t0 - 16)
                m1lo = lane == t1
                m1hi = lane == (t1 - 16)
                w0lo = jnp.where(m0lo, w0s, w0lo)
                w0hi = jnp.where(m0hi, w0s, w0hi)
                v0lo = jnp.where(m0lo, vs, v0lo)
                v0hi = jnp.where(m0hi, vs, v0hi)
                w1lo = jnp.where(m1lo, w0s, w1lo)
                w1hi = jnp.where(m1hi, w0s, w1hi)
                v1lo = jnp.where(m1lo, vs, v1lo)
                v1hi = jnp.where(m1hi, vs, v1hi)
                inc = jnp.where(loc, 1, 0)
                fc0 = fc0 + inc
                fc1 = fc1 + (1 - inc)
            fl0 = fc0 >= 16

            @pl.when(fl0)
            def _(w0lo=w0lo, v0lo=v0lo, off0=off0):
                sw0[0][pl.ds(off0, 16)] = w0lo
                sv[0][pl.ds(off0, 16)] = v0lo

            w0lo = jnp.where(fl0, w0hi, w0lo)
            v0lo = jnp.where(fl0, v0hi, v0lo)
            fc0 = jnp.where(fl0, fc0 - 16, fc0)
            off0 = jnp.where(fl0, off0 + 16, off0)
            fl1 = fc1 >= 16

            @pl.when(fl1)
            def _(w1lo=w1lo, v1lo=v1lo, off1=off1):
                sw0[1][pl.ds(off1, 16)] = w1lo
                sv[1][pl.ds(off1, 16)] = v1lo

            w1lo = jnp.where(fl1, w1hi, w1lo)
            v1lo = jnp.where(fl1, v1hi, v1lo)
            fc1 = jnp.where(fl1, fc1 - 16, fc1)
            off1 = jnp.where(fl1, off1 + 16, off1)
        if next_j is not None:
            meta_issue(next_j, bufs)
        return (fc0, off0, w0lo, w0hi, v0lo, v0hi,
                fc1, off1, w1lo, w1hi, v1lo, v1hi)

    A = (irow, icol, ival, semA)
    Bb = (irow2, icol2, ival2, semB)
    NCH_IN = EPB // IC

    def pair(k, carry):
        carry = process(carry, A, 2 * k + 2)
        carry = process(carry, Bb, 2 * k + 3)
        return carry

    meta_issue(0, A)
    meta_issue(1, Bb)
    init = (jnp.int32(0), jnp.int32(0), zi, zi, zf, zf,
            jnp.int32(0), jnp.int32(0), zi, zi, zf, zf)
    carry = lax.fori_loop(0, NCH_IN // 2, pair, init)
    carry = process(carry, A, None)
    meta_drain(Bb)
    (fc0, off0, w0lo, _, v0lo, _,
     fc1, off1, w1lo, _, v1lo, _) = carry

    trash_w0 = jnp.broadcast_to((HALF + (w % NS)) << 14, (16,)).astype(jnp.int32)
    for h, fc, off, wlo, vlo in ((0, fc0, off0, w0lo, v0lo),
                                 (1, fc1, off1, w1lo, v1lo)):
        sw0[h][pl.ds(off, 16)] = jnp.where(lane < fc, wlo, trash_w0)
        sv[h][pl.ds(off, 16)] = jnp.where(lane < fc, vlo, zf)
        sw0[h][pl.ds(off + 16, 16)] = trash_w0
        sv[h][pl.ds(off + 16, 16)] = zf
        sw0[h][pl.ds(off + 32, 16)] = trash_w0
        sv[h][pl.ds(off + 32, 16)] = zf
        base = h * HSTRIDE + w * RCAP
        pltpu.sync_copy(sw0[h], bw0_out.at[pl.ds(base, RCAP)])
        pltpu.sync_copy(sv[h], bval_out.at[pl.ds(base, RCAP)])
        nch = (off + fc + (EC - 1)) // EC
        cbuf[pl.ds(0, 16)] = jnp.where(lane == 0, nch, 0)
        pltpu.sync_copy(cbuf, counts_out.at[pl.ds((h * NW + w) * 16, 16)])


_bucket = pl.kernel(
    _bucket_body,
    out_type=(jax.ShapeDtypeStruct((BLEN,), jnp.int32),
              jax.ShapeDtypeStruct((BLEN,), jnp.float32),
              jax.ShapeDtypeStruct((NC * NW * 16,), jnp.int32)),
    mesh=plsc.VectorSubcoreMesh(core_axis_name="c", subcore_axis_name="s"),
    scratch_types=[
        pltpu.VMEM((IC,), jnp.int32),
        pltpu.VMEM((IC,), jnp.int32),
        pltpu.VMEM((IC,), jnp.float32),
        pltpu.VMEM((IC,), jnp.int32),
        pltpu.VMEM((IC,), jnp.int32),
        pltpu.VMEM((IC,), jnp.float32),
        pltpu.VMEM((RCAP,), jnp.int32),
        pltpu.VMEM((RCAP,), jnp.float32),
        pltpu.VMEM((RCAP,), jnp.int32),
        pltpu.VMEM((RCAP,), jnp.float32),
        pltpu.VMEM((16,), jnp.int32),
        pltpu.SemaphoreType.DMA,
        pltpu.SemaphoreType.DMA,
    ],
)


def _hop_body(x_hbm, bw0, bval, counts, zero_hbm, out_hbm,
              x_sh, acc_sh, cbuf, tridx,
              w0A, lrowA, colA, valA, rowsA,
              w0B, lrowB, colB, valB, rowsB,
              semMA, semMB, semGA, semGB, semSA, semSB):
    c = lax.axis_index("c")
    s = lax.axis_index("s")
    lo = c * HALF
    trash = HALF + s

    pltpu.sync_copy(x_hbm.at[pl.ds(s * XROWS_PER_SUB, XROWS_PER_SUB)],
                    x_sh.at[pl.ds(s * XROWS_PER_SUB, XROWS_PER_SUB)])

    @pl.when(s == 0)
    def _():
        pltpu.sync_copy(x_hbm.at[pl.ds(NS * XROWS_PER_SUB, XTAIL)],
                        x_sh.at[pl.ds(NS * XROWS_PER_SUB, XTAIL)])

    @pl.when(s == 1)
    def _():
        pltpu.sync_copy(zero_hbm, acc_sh)

    for t in range(EC // 16):
        tridx[pl.ds(t * 16, 16)] = jnp.broadcast_to(trash, (16,)).astype(jnp.int32)
    plsc.subcore_barrier()

    A = (w0A, lrowA, colA, valA, rowsA, semMA, semGA, semSA)
    Bb = (w0B, lrowB, colB, valB, rowsB, semMB, semGB, semSB)

    def meta_issue(rbase, j, bufs):
        w0b, valb, semM = bufs[0], bufs[3], bufs[5]
        off = rbase + j * EC
        pltpu.async_copy(bw0.at[pl.ds(off, EC)], w0b, semM)
        pltpu.async_copy(bval.at[pl.ds(off, EC)], valb, semM)

    def meta_drain(rbase, bufs):
        w0b, valb, semM = bufs[0], bufs[3], bufs[5]
        pltpu.make_async_copy(bw0.at[pl.ds(rbase, EC)], w0b, semM).wait()
        pltpu.make_async_copy(bval.at[pl.ds(rbase, EC)], valb, semM).wait()

    def start_phase(rbase, bufs):
        w0b, lrowb, colb, rowsb = bufs[0], bufs[1], bufs[2], bufs[4]
        semG, semS = bufs[6], bufs[7]
        meta_drain(rbase, bufs)
        for g in range(EC // 16):
            wv = w0b[pl.ds(g * 16, 16)]
            lrowb[pl.ds(g * 16, 16)] = lax.shift_right_logical(wv, 14)
            colb[pl.ds(g * 16, 16)] = wv & 16383
        pltpu.make_async_copy(rowsb, acc_sh.at[lrowb], semS).wait()
        pltpu.async_copy(x_sh.at[colb], rowsb, semG)

    def finish_phase(rbase, bufs, next_meta_j):
        lrowb, colb, valb, rowsb = bufs[1], bufs[2], bufs[3], bufs[4]
        semG, semS = bufs[6], bufs[7]
        pltpu.make_async_copy(x_sh.at[colb], rowsb, semG).wait()

        def scale16(g, _):
            vv = valb[pl.ds(g * 16, 16)]
            for l in range(16):
                e = g * 16 + l
                v = vv[l]
                for q in range(D // 16):
                    rowsb[e, pl.ds(q * 16, 16)] = rowsb[e, pl.ds(q * 16, 16)] * v
            return 0

        lax.fori_loop(0, EC // 16, scale16, 0)
        pltpu.async_copy(rowsb, acc_sh.at[lrowb], semS)
        if next_meta_j is not None:
            meta_issue(rbase, next_meta_j, bufs)

    def do_region(ridx, nch):
        rbase = c * HSTRIDE + ridx * RCAP
        pltpu.async_copy(rowsA, acc_sh.at[tridx], semSA)
        pltpu.async_copy(rowsB, acc_sh.at[tridx], semSB)
        meta_issue(rbase, 0, A)
        meta_issue(rbase, 1, Bb)

        def pair(k, carry):
            start_phase(rbase, A)
            start_phase(rbase, Bb)
            finish_phase(rbase, A, 2 * k + 2)
            finish_phase(rbase, Bb, 2 * k + 3)
            return carry

        lax.fori_loop(0, nch // 2, pair, 0)
        odd = nch & 1

        @pl.when(odd == 1)
        def _():
            start_phase(rbase, A)
            finish_phase(rbase, A, None)

        pltpu.make_async_copy(rowsA, acc_sh.at[lrowA], semSA).wait()
        pltpu.make_async_copy(rowsB, acc_sh.at[lrowB], semSB).wait()
        meta_drain(rbase, Bb)

        @pl.when(odd == 0)
        def _():
            meta_drain(rbase, A)

    pltpu.sync_copy(counts.at[pl.ds((c * NW + 2 * s) * 16, 32)], cbuf)
    n0 = cbuf[pl.ds(0, 16)][0]
    n1 = cbuf[pl.ds(16, 16)][0]
    do_region(2 * s, n0)
    do_region(2 * s + 1, n1)
    plsc.subcore_barrier()

    @pl.when(s == 0)
    def _():
        pltpu.sync_copy(acc_sh.at[pl.ds(0, HALF)], out_hbm.at[pl.ds(lo, HALF)])


_hop = pl.kernel(
    _hop_body,
    out_type=jax.ShapeDtypeStruct((NN, D), jnp.float32),
    mesh=plsc.VectorSubcoreMesh(core_axis_name="c", subcore_axis_name="s"),
    scratch_types=[
        pltpu.VMEM_SHARED((NN, D), jnp.float32),
        pltpu.VMEM_SHARED((ACC_ROWS, D), jnp.float32),
        pltpu.VMEM((32,), jnp.int32),
        pltpu.VMEM((EC,), jnp.int32),
        pltpu.VMEM((EC,), jnp.int32),
        pltpu.VMEM((EC,), jnp.int32),
        pltpu.VMEM((EC,), jnp.int32),
        pltpu.VMEM((EC,), jnp.float32),
        pltpu.VMEM((EC, D), jnp.float32),
        pltpu.VMEM((EC,), jnp.int32),
        pltpu.VMEM((EC,), jnp.int32),
        pltpu.VMEM((EC,), jnp.int32),
        pltpu.VMEM((EC,), jnp.float32),
        pltpu.VMEM((EC, D), jnp.float32),
        pltpu.SemaphoreType.DMA,
        pltpu.SemaphoreType.DMA,
        pltpu.SemaphoreType.DMA,
        pltpu.SemaphoreType.DMA,
        pltpu.SemaphoreType.DMA,
        pltpu.SemaphoreType.DMA,
    ],
)

BPW = BATCH // NW


def _loss_body(x0, x1, x2, x3, u_hbm, i_hbm, n_hbm, pos_out, neg_out,
               idxv, tmp, usum, isum, ng0, ng1, ng2, ng3, pos_sm, neg_sm, sem):
    c = lax.axis_index("c")
    s = lax.axis_index("s")
    w = s * NC + c
    b0 = w * BPW
    tables = (x0, x1, x2, x3)
    ngs = (ng0, ng1, ng2, ng3)

    def gather_sum(idx_hbm, off, dst):
        pltpu.sync_copy(idx_hbm.at[pl.ds(off, BPW)], idxv)
        pltpu.async_copy(tables[0].at[idxv], dst, sem).wait()
        for t in range(1, 4):
            pltpu.async_copy(tables[t].at[idxv], tmp, sem).wait()

            def addloop(i, _):
                for q in range(D // 16):
                    dst[i, pl.ds(q * 16, 16)] = (dst[i, pl.ds(q * 16, 16)]
                                                 + tmp[i, pl.ds(q * 16, 16)])
                return 0

            lax.fori_loop(0, BPW, addloop, 0)

    gather_sum(u_hbm, b0, usum)
    gather_sum(i_hbm, b0, isum)
    for n in range(NNEG):
        gather_sum(n_hbm, n * BATCH + b0, ngs[n])

    def dots(b, _):
        pacc = jnp.zeros((16,), jnp.float32)
        for q in range(D // 16):
            pacc = pacc + (usum[b, pl.ds(q * 16, 16)]
                           * isum[b, pl.ds(q * 16, 16)])
        pos_sm[b, pl.ds(0, 16)] = pacc
        for n in range(NNEG):
            nacc = jnp.zeros((16,), jnp.float32)
            for q in range(D // 16):
                nacc = nacc + (usum[b, pl.ds(q * 16, 16)]
                               * ngs[n][b, pl.ds(q * 16, 16)])
            neg_sm[n * BPW + b, pl.ds(0, 16)] = nacc
        return 0

    lax.fori_loop(0, BPW, dots, 0)
    pltpu.sync_copy(pos_sm, pos_out.at[pl.ds(b0, BPW)])
    for n in range(NNEG):
        pltpu.sync_copy(neg_sm.at[pl.ds(n * BPW, BPW)],
                        neg_out.at[pl.ds(n * BATCH + b0, BPW)])


_loss = pl.kernel(
    _loss_body,
    out_type=(jax.ShapeDtypeStruct((BATCH, 16), jnp.float32),
              jax.ShapeDtypeStruct((NNEG * BATCH, 16), jnp.float32)),
    mesh=plsc.VectorSubcoreMesh(core_axis_name="c", subcore_axis_name="s"),
    scratch_types=[
        pltpu.VMEM((BPW,), jnp.int32),
        pltpu.VMEM((BPW, D), jnp.float32),
        pltpu.VMEM((BPW, D), jnp.float32),
        pltpu.VMEM((BPW, D), jnp.float32),
        pltpu.VMEM((BPW, D), jnp.float32),
        pltpu.VMEM((BPW, D), jnp.float32),
        pltpu.VMEM((BPW, D), jnp.float32),
        pltpu.VMEM((BPW, D), jnp.float32),
        pltpu.VMEM((BPW, 16), jnp.float32),
        pltpu.VMEM((NNEG * BPW, 16), jnp.float32),
        pltpu.SemaphoreType.DMA,
    ],
)


def _nce_body(p_ref, n_ref, o_ref):
    p = jnp.sum(p_ref[...], axis=-1) * (1.0 / 16.0)
    nk = jnp.sum(n_ref[...], axis=-1) * (1.0 / 16.0)
    ne = jnp.sum(jnp.exp(nk), axis=0)
    loss = jnp.mean(jnp.log(jnp.exp(p) + ne) - p)
    o_ref[...] = jnp.reshape(loss, (1, 1))


_nce = pl.pallas_call(
    _nce_body,
    out_shape=jax.ShapeDtypeStruct((1, 1), jnp.float32),
)


def kernel(edge_vals, user_emb, item_emb, users, items, negatives, edge_index):
    all_emb = jnp.concatenate([user_emb, item_emb], axis=0).astype(jnp.float32)
    padi = jnp.zeros((IC,), jnp.int32)
    row = jnp.concatenate([edge_index[0].astype(jnp.int32), padi])
    col = jnp.concatenate([edge_index[1].astype(jnp.int32), padi])
    ev = jnp.concatenate([edge_vals.astype(jnp.float32),
                          jnp.zeros((IC,), jnp.float32)])
    zero_acc = jnp.zeros((ACC_ROWS, D), jnp.float32)

    bw0, bval, counts = _bucket(row, col, ev)

    x0 = all_emb
    x1 = _hop(x0, bw0, bval, counts, zero_acc)
    x2 = _hop(x1, bw0, bval, counts, zero_acc)
    x3 = _hop(x2, bw0, bval, counts, zero_acc)

    u = users.astype(jnp.int32)
    it = items.astype(jnp.int32) + N_USERS
    ng = negatives.astype(jnp.int32) + N_USERS
    pos, negk = _loss(x0, x1, x2, x3, u, it, ng)
    out = _nce(pos, negk.reshape(NNEG, BATCH, 16))
    return out[0, 0]


# speedup vs baseline: 1.9856x; 1.0310x over previous
import jax
import jax.numpy as jnp
from jax import lax
from jax.experimental import pallas as pl
from jax.experimental.pallas import tpu as pltpu
from jax.experimental.pallas import tpu_sc as plsc

N_USERS = 2000
N_ITEMS = 8000
NN = N_USERS + N_ITEMS
D = 128
HOPS = 3
NNEG = 4
BATCH = 1024
E = 320000

NC = 2
NS = 16
NW = NC * NS
HALF = NN // NC
ACC_ROWS = 5024
EC = 32

EPB = E // NW
IC = 80
RCAP = 10048
RCH = RCAP // EC
HSTRIDE = NW * RCAP
BLEN = NC * HSTRIDE + EC

XROWS_PER_SUB = 624
XTAIL = NN - NS * XROWS_PER_SUB


def _bucket_body(row_hbm, col_hbm, val_hbm,
                 bw0_out, bval_out, counts_out,
                 irow, icol, ival, irow2, icol2, ival2,
                 sw0_0, sv_0, sw0_1, sv_1, cbuf, semA, semB):
    c = lax.axis_index("c")
    s = lax.axis_index("s")
    w = s * NC + c
    ebase = w * EPB
    lane = lax.iota(jnp.int32, 16)
    sw0 = (sw0_0, sw0_1)
    sv = (sv_0, sv_1)
    zi = jnp.zeros((16,), jnp.int32)
    zf = jnp.zeros((16,), jnp.float32)

    def meta_issue(j, bufs):
        irowb, icolb, ivalb, semM = bufs
        off = ebase + j * IC
        pltpu.async_copy(row_hbm.at[pl.ds(off, IC)], irowb, semM)
        pltpu.async_copy(col_hbm.at[pl.ds(off, IC)], icolb, semM)
        pltpu.async_copy(val_hbm.at[pl.ds(off, IC)], ivalb, semM)

    def meta_drain(bufs):
        irowb, icolb, ivalb, semM = bufs
        pltpu.make_async_copy(row_hbm.at[pl.ds(0, IC)], irowb, semM).wait()
        pltpu.make_async_copy(col_hbm.at[pl.ds(0, IC)], icolb, semM).wait()
        pltpu.make_async_copy(val_hbm.at[pl.ds(0, IC)], ivalb, semM).wait()

    def process(carry, bufs, next_j):
        (fc0, off0, w0lo, w0hi, v0lo, v0hi,
         fc1, off1, w1lo, w1hi, v1lo, v1hi) = carry
        irowb, icolb, ivalb = bufs[0], bufs[1], bufs[2]
        meta_drain(bufs)
        for g in range(IC // 16):
            r = irowb[pl.ds(g * 16, 16)]
            cv = icolb[pl.ds(g * 16, 16)]
            vv = ivalb[pl.ds(g * 16, 16)]
            lr = jnp.where(r >= HALF, r - HALF, r)
            w0v = lax.shift_left(lr, 14) + cv
            for l in range(16):
                w0s = w0v[l]
                vs = vv[l]
                loc = r[l] < HALF
                t0 = jnp.where(loc, fc0, -1)
                t1 = jnp.where(loc, -1, fc1)
                m0lo = lane == t0
                m0hi = lane == (t0 - 16)
                m1lo = lane == t1
                m1hi = lane == (t1 - 16)
                w0lo = jnp.where(m0lo, w0s, w0lo)
                w0hi = jnp.where(m0hi, w0s, w0hi)
                v0lo = jnp.where(m0lo, vs, v0lo)
                v0hi = jnp.where(m0hi, vs, v0hi)
                w1lo = jnp.where(m1lo, w0s, w1lo)
                w1hi = jnp.where(m1hi, w0s, w1hi)
                v1lo = jnp.where(m1lo, vs, v1lo)
                v1hi = jnp.where(m1hi, vs, v1hi)
                inc = jnp.where(loc, 1, 0)
                fc0 = fc0 + inc
                fc1 = fc1 + (1 - inc)
            fl0 = fc0 >= 16

            @pl.when(fl0)
            def _(w0lo=w0lo, v0lo=v0lo, off0=off0):
                sw0[0][pl.ds(off0, 16)] = w0lo
                sv[0][pl.ds(off0, 16)] = v0lo

            w0lo = jnp.where(fl0, w0hi, w0lo)
            v0lo = jnp.where(fl0, v0hi, v0lo)
            fc0 = jnp.where(fl0, fc0 - 16, fc0)
            off0 = jnp.where(fl0, off0 + 16, off0)
            fl1 = fc1 >= 16

            @pl.when(fl1)
            def _(w1lo=w1lo, v1lo=v1lo, off1=off1):
                sw0[1][pl.ds(off1, 16)] = w1lo
                sv[1][pl.ds(off1, 16)] = v1lo

            w1lo = jnp.where(fl1, w1hi, w1lo)
            v1lo = jnp.where(fl1, v1hi, v1lo)
            fc1 = jnp.where(fl1, fc1 - 16, fc1)
            off1 = jnp.where(fl1, off1 + 16, off1)
        if next_j is not None:
            meta_issue(next_j, bufs)
        return (fc0, off0, w0lo, w0hi, v0lo, v0hi,
                fc1, off1, w1lo, w1hi, v1lo, v1hi)

    A = (irow, icol, ival, semA)
    Bb = (irow2, icol2, ival2, semB)
    NCH_IN = EPB // IC

    def pair(k, carry):
        carry = process(carry, A, 2 * k + 2)
        carry = process(carry, Bb, 2 * k + 3)
        return carry

    meta_issue(0, A)
    meta_issue(1, Bb)
    init = (jnp.int32(0), jnp.int32(0), zi, zi, zf, zf,
            jnp.int32(0), jnp.int32(0), zi, zi, zf, zf)
    carry = lax.fori_loop(0, NCH_IN // 2, pair, init)
    carry = process(carry, A, None)
    meta_drain(Bb)
    (fc0, off0, w0lo, _, v0lo, _,
     fc1, off1, w1lo, _, v1lo, _) = carry

    trash_w0 = jnp.broadcast_to((HALF + (w % NS)) << 14, (16,)).astype(jnp.int32)
    for h, fc, off, wlo, vlo in ((0, fc0, off0, w0lo, v0lo),
                                 (1, fc1, off1, w1lo, v1lo)):
        sw0[h][pl.ds(off, 16)] = jnp.where(lane < fc, wlo, trash_w0)
        sv[h][pl.ds(off, 16)] = jnp.where(lane < fc, vlo, zf)
        sw0[h][pl.ds(off + 16, 16)] = trash_w0
        sv[h][pl.ds(off + 16, 16)] = zf
        sw0[h][pl.ds(off + 32, 16)] = trash_w0
        sv[h][pl.ds(off + 32, 16)] = zf
        base = h * HSTRIDE + w * RCAP
        pltpu.sync_copy(sw0[h], bw0_out.at[pl.ds(base, RCAP)])
        pltpu.sync_copy(sv[h], bval_out.at[pl.ds(base, RCAP)])
        nch = (off + fc + (EC - 1)) // EC
        cbuf[pl.ds(0, 16)] = jnp.where(lane == 0, nch, 0)
        pltpu.sync_copy(cbuf, counts_out.at[pl.ds((h * NW + w) * 16, 16)])


_bucket = pl.kernel(
    _bucket_body,
    out_type=(jax.ShapeDtypeStruct((BLEN,), jnp.int32),
              jax.ShapeDtypeStruct((BLEN,), jnp.float32),
              jax.ShapeDtypeStruct((NC * NW * 16,), jnp.int32)),
    mesh=plsc.VectorSubcoreMesh(core_axis_name="c", subcore_axis_name="s"),
    scratch_types=[
        pltpu.VMEM((IC,), jnp.int32),
        pltpu.VMEM((IC,), jnp.int32),
        pltpu.VMEM((IC,), jnp.float32),
        pltpu.VMEM((IC,), jnp.int32),
        pltpu.VMEM((IC,), jnp.int32),
        pltpu.VMEM((IC,), jnp.float32),
        pltpu.VMEM((RCAP,), jnp.int32),
        pltpu.VMEM((RCAP,), jnp.float32),
        pltpu.VMEM((RCAP,), jnp.int32),
        pltpu.VMEM((RCAP,), jnp.float32),
        pltpu.VMEM((16,), jnp.int32),
        pltpu.SemaphoreType.DMA,
        pltpu.SemaphoreType.DMA,
    ],
)


def _hop_body(x_hbm, bw0, bval, counts, zero_hbm, out_hbm,
              x_sh, acc_sh, cbuf, tridx,
              w0A, lrowA, colA, valA, rowsA,
              w0B, lrowB, colB, valB, rowsB,
              semMA, semMB, semGA, semGB, semSA, semSB):
    c = lax.axis_index("c")
    s = lax.axis_index("s")
    lo = c * HALF
    trash = HALF + s

    pltpu.sync_copy(x_hbm.at[pl.ds(s * XROWS_PER_SUB, XROWS_PER_SUB)],
                    x_sh.at[pl.ds(s * XROWS_PER_SUB, XROWS_PER_SUB)])

    @pl.when(s == 0)
    def _():
        pltpu.sync_copy(x_hbm.at[pl.ds(NS * XROWS_PER_SUB, XTAIL)],
                        x_sh.at[pl.ds(NS * XROWS_PER_SUB, XTAIL)])

    @pl.when(s == 1)
    def _():
        pltpu.sync_copy(zero_hbm, acc_sh)

    for t in range(EC // 16):
        tridx[pl.ds(t * 16, 16)] = jnp.broadcast_to(trash, (16,)).astype(jnp.int32)
    plsc.subcore_barrier()

    A = (w0A, lrowA, colA, valA, rowsA, semMA, semGA, semSA)
    Bb = (w0B, lrowB, colB, valB, rowsB, semMB, semGB, semSB)

    def meta_issue(rbase, j, bufs):
        w0b, valb, semM = bufs[0], bufs[3], bufs[5]
        off = rbase + j * EC
        pltpu.async_copy(bw0.at[pl.ds(off, EC)], w0b, semM)
        pltpu.async_copy(bval.at[pl.ds(off, EC)], valb, semM)

    def meta_drain(rbase, bufs):
        w0b, valb, semM = bufs[0], bufs[3], bufs[5]
        pltpu.make_async_copy(bw0.at[pl.ds(rbase, EC)], w0b, semM).wait()
        pltpu.make_async_copy(bval.at[pl.ds(rbase, EC)], valb, semM).wait()

    def start_phase(rbase, bufs):
        w0b, lrowb, colb, rowsb = bufs[0], bufs[1], bufs[2], bufs[4]
        semG, semS = bufs[6], bufs[7]
        meta_drain(rbase, bufs)
        for g in range(EC // 16):
            wv = w0b[pl.ds(g * 16, 16)]
            lrowb[pl.ds(g * 16, 16)] = lax.shift_right_logical(wv, 14)
            colb[pl.ds(g * 16, 16)] = wv & 16383
        pltpu.make_async_copy(rowsb, acc_sh.at[lrowb], semS).wait()
        pltpu.async_copy(x_sh.at[colb], rowsb, semG)

    def finish_phase(rbase, bufs, next_meta_j):
        lrowb, colb, valb, rowsb = bufs[1], bufs[2], bufs[3], bufs[4]
        semG, semS = bufs[6], bufs[7]
        pltpu.make_async_copy(x_sh.at[colb], rowsb, semG).wait()

        def scale16(g, _):
            vv = valb[pl.ds(g * 16, 16)]
            for l in range(16):
                e = g * 16 + l
                v = vv[l]
                for q in range(D // 16):
                    rowsb[e, pl.ds(q * 16, 16)] = rowsb[e, pl.ds(q * 16, 16)] * v
            return 0

        lax.fori_loop(0, EC // 16, scale16, 0)
        pltpu.async_copy(rowsb, acc_sh.at[lrowb], semS)
        if next_meta_j is not None:
            meta_issue(rbase, next_meta_j, bufs)

    def do_region(ridx, nch):
        rbase = c * HSTRIDE + ridx * RCAP
        pltpu.async_copy(rowsA, acc_sh.at[tridx], semSA)
        pltpu.async_copy(rowsB, acc_sh.at[tridx], semSB)
        meta_issue(rbase, 0, A)
        meta_issue(rbase, 1, Bb)

        def pair(k, carry):
            start_phase(rbase, A)
            start_phase(rbase, Bb)
            finish_phase(rbase, A, 2 * k + 2)
            finish_phase(rbase, Bb, 2 * k + 3)
            return carry

        lax.fori_loop(0, nch // 2, pair, 0)
        odd = nch & 1

        @pl.when(odd == 1)
        def _():
            start_phase(rbase, A)
            finish_phase(rbase, A, None)

        pltpu.make_async_copy(rowsA, acc_sh.at[lrowA], semSA).wait()
        pltpu.make_async_copy(rowsB, acc_sh.at[lrowB], semSB).wait()
        meta_drain(rbase, Bb)

        @pl.when(odd == 0)
        def _():
            meta_drain(rbase, A)

    pltpu.sync_copy(counts.at[pl.ds((c * NW + 2 * s) * 16, 32)], cbuf)
    n0 = cbuf[pl.ds(0, 16)][0]
    n1 = cbuf[pl.ds(16, 16)][0]
    do_region(2 * s, n0)
    do_region(2 * s + 1, n1)
    plsc.subcore_barrier()

    @pl.when(s == 0)
    def _():
        pltpu.sync_copy(acc_sh.at[pl.ds(0, HALF)], out_hbm.at[pl.ds(lo, HALF)])


_hop = pl.kernel(
    _hop_body,
    out_type=jax.ShapeDtypeStruct((NN, D), jnp.float32),
    mesh=plsc.VectorSubcoreMesh(core_axis_name="c", subcore_axis_name="s"),
    scratch_types=[
        pltpu.VMEM_SHARED((NN, D), jnp.float32),
        pltpu.VMEM_SHARED((ACC_ROWS, D), jnp.float32),
        pltpu.VMEM((32,), jnp.int32),
        pltpu.VMEM((EC,), jnp.int32),
        pltpu.VMEM((EC,), jnp.int32),
        pltpu.VMEM((EC,), jnp.int32),
        pltpu.VMEM((EC,), jnp.int32),
        pltpu.VMEM((EC,), jnp.float32),
        pltpu.VMEM((EC, D), jnp.float32),
        pltpu.VMEM((EC,), jnp.int32),
        pltpu.VMEM((EC,), jnp.int32),
        pltpu.VMEM((EC,), jnp.int32),
        pltpu.VMEM((EC,), jnp.float32),
        pltpu.VMEM((EC, D), jnp.float32),
        pltpu.SemaphoreType.DMA,
        pltpu.SemaphoreType.DMA,
        pltpu.SemaphoreType.DMA,
        pltpu.SemaphoreType.DMA,
        pltpu.SemaphoreType.DMA,
        pltpu.SemaphoreType.DMA,
    ],
)

BPW = BATCH // NW


def _loss_body(x0, x1, x2, x3, u_hbm, i_hbm, n_hbm, pos_out, neg_out,
               uidx, iidx, nidx, bufs_flat, pos_sm, neg_sm, sem):
    c = lax.axis_index("c")
    s = lax.axis_index("s")
    w = s * NC + c
    b0 = w * BPW
    tables = (x0, x1, x2, x3)
    dsts = bufs_flat[0:6]
    tsets = (bufs_flat[6:12], bufs_flat[12:18], bufs_flat[18:24])
    usum, isum, ng0, ng1, ng2, ng3 = dsts
    ngs = (ng0, ng1, ng2, ng3)

    pltpu.async_copy(u_hbm.at[pl.ds(b0, BPW)], uidx, sem)
    pltpu.async_copy(i_hbm.at[pl.ds(b0, BPW)], iidx, sem)
    for n in range(NNEG):
        pltpu.async_copy(n_hbm.at[pl.ds(n * BATCH + b0, BPW)],
                         nidx.at[pl.ds(n * BPW, BPW)], sem)
    pltpu.make_async_copy(u_hbm.at[pl.ds(b0, BPW)], uidx, sem).wait()
    pltpu.make_async_copy(i_hbm.at[pl.ds(b0, BPW)], iidx, sem).wait()
    for n in range(NNEG):
        pltpu.make_async_copy(n_hbm.at[pl.ds(n * BATCH + b0, BPW)],
                              nidx.at[pl.ds(n * BPW, BPW)], sem).wait()

    idxs = (uidx.at[pl.ds(0, BPW)], iidx.at[pl.ds(0, BPW)],
            nidx.at[pl.ds(0, BPW)], nidx.at[pl.ds(BPW, BPW)],
            nidx.at[pl.ds(2 * BPW, BPW)], nidx.at[pl.ds(3 * BPW, BPW)])
    for r in range(6):
        pltpu.async_copy(tables[0].at[idxs[r]], dsts[r], sem)
    for t in range(1, 4):
        for r in range(6):
            pltpu.async_copy(tables[t].at[idxs[r]], tsets[t - 1][r], sem)
    for r in range(6):
        pltpu.make_async_copy(tables[0].at[idxs[r]], dsts[r], sem).wait()
    for t in range(1, 4):
        for r in range(6):
            pltpu.make_async_copy(tables[t].at[idxs[r]], tsets[t - 1][r],
                                  sem).wait()

    for t in range(1, 4):
        for r in range(6):

            def addloop(i, _, t=t, r=r):
                for q in range(D // 16):
                    dsts[r][i, pl.ds(q * 16, 16)] = (
                        dsts[r][i, pl.ds(q * 16, 16)]
                        + tsets[t - 1][r][i, pl.ds(q * 16, 16)])
                return 0

            lax.fori_loop(0, BPW, addloop, 0)

    def dots(b, _):
        pacc = jnp.zeros((16,), jnp.float32)
        for q in range(D // 16):
            pacc = pacc + (usum[b, pl.ds(q * 16, 16)]
                           * isum[b, pl.ds(q * 16, 16)])
        pos_sm[b, pl.ds(0, 16)] = pacc
        for n in range(NNEG):
            nacc = jnp.zeros((16,), jnp.float32)
            for q in range(D // 16):
                nacc = nacc + (usum[b, pl.ds(q * 16, 16)]
                               * ngs[n][b, pl.ds(q * 16, 16)])
            neg_sm[n * BPW + b, pl.ds(0, 16)] = nacc
        return 0

    lax.fori_loop(0, BPW, dots, 0)
    pltpu.sync_copy(pos_sm, pos_out.at[pl.ds(b0, BPW)])
    for n in range(NNEG):
        pltpu.sync_copy(neg_sm.at[pl.ds(n * BPW, BPW)],
                        neg_out.at[pl.ds(n * BATCH + b0, BPW)])


_loss = pl.kernel(
    _loss_body,
    out_type=(jax.ShapeDtypeStruct((BATCH, 16), jnp.float32),
              jax.ShapeDtypeStruct((NNEG * BATCH, 16), jnp.float32)),
    mesh=plsc.VectorSubcoreMesh(core_axis_name="c", subcore_axis_name="s"),
    scratch_types=[
        pltpu.VMEM((BPW,), jnp.int32),
        pltpu.VMEM((BPW,), jnp.int32),
        pltpu.VMEM((NNEG * BPW,), jnp.int32),
        [pltpu.VMEM((BPW, D), jnp.float32) for _ in range(24)],
        pltpu.VMEM((BPW, 16), jnp.float32),
        pltpu.VMEM((NNEG * BPW, 16), jnp.float32),
        pltpu.SemaphoreType.DMA,
    ],
)


def _nce_body(p_ref, n_ref, o_ref):
    p = jnp.sum(p_ref[...], axis=-1) * (1.0 / 16.0)
    nk = jnp.sum(n_ref[...], axis=-1) * (1.0 / 16.0)
    ne = jnp.sum(jnp.exp(nk), axis=0)
    loss = jnp.mean(jnp.log(jnp.exp(p) + ne) - p)
    o_ref[...] = jnp.reshape(loss, (1, 1))


_nce = pl.pallas_call(
    _nce_body,
    out_shape=jax.ShapeDtypeStruct((1, 1), jnp.float32),
)


def kernel(edge_vals, user_emb, item_emb, users, items, negatives, edge_index):
    all_emb = jnp.concatenate([user_emb, item_emb], axis=0).astype(jnp.float32)
    padi = jnp.zeros((IC,), jnp.int32)
    row = jnp.concatenate([edge_index[0].astype(jnp.int32), padi])
    col = jnp.concatenate([edge_index[1].astype(jnp.int32), padi])
    ev = jnp.concatenate([edge_vals.astype(jnp.float32),
                          jnp.zeros((IC,), jnp.float32)])
    zero_acc = jnp.zeros((ACC_ROWS, D), jnp.float32)

    bw0, bval, counts = _bucket(row, col, ev)

    x0 = all_emb
    x1 = _hop(x0, bw0, bval, counts, zero_acc)
    x2 = _hop(x1, bw0, bval, counts, zero_acc)
    x3 = _hop(x2, bw0, bval, counts, zero_acc)

    u = users.astype(jnp.int32)
    it = items.astype(jnp.int32) + N_USERS
    ng = negatives.astype(jnp.int32) + N_USERS
    pos, negk = _loss(x0, x1, x2, x3, u, it, ng)
    out = _nce(pos, negk.reshape(NNEG, BATCH, 16))
    return out[0, 0]


# speedup vs baseline: 1.9939x; 1.0042x over previous
import jax
import jax.numpy as jnp
from jax import lax
from jax.experimental import pallas as pl
from jax.experimental.pallas import tpu as pltpu
from jax.experimental.pallas import tpu_sc as plsc

N_USERS = 2000
N_ITEMS = 8000
NN = N_USERS + N_ITEMS
D = 128
HOPS = 3
NNEG = 4
BATCH = 1024
E = 320000

NC = 2
NS = 16
NW = NC * NS
HALF = NN // NC
ACC_ROWS = 5024
EC = 32

EPB = E // NW
IC = 80
RCAP = 10048
RCH = RCAP // EC
HSTRIDE = NW * RCAP
BLEN = NC * HSTRIDE + 8 * EC

XROWS_PER_SUB = 624
XTAIL = NN - NS * XROWS_PER_SUB


def _bucket_body(row_hbm, col_hbm, val_hbm,
                 bw0_out, bval_out, counts_out,
                 irow, icol, ival, irow2, icol2, ival2,
                 sw0_0, sv_0, sw0_1, sv_1, cbuf, semA, semB):
    c = lax.axis_index("c")
    s = lax.axis_index("s")
    w = s * NC + c
    ebase = w * EPB
    lane = lax.iota(jnp.int32, 16)
    sw0 = (sw0_0, sw0_1)
    sv = (sv_0, sv_1)
    zi = jnp.zeros((16,), jnp.int32)
    zf = jnp.zeros((16,), jnp.float32)

    def meta_issue(j, bufs):
        irowb, icolb, ivalb, semM = bufs
        off = ebase + j * IC
        pltpu.async_copy(row_hbm.at[pl.ds(off, IC)], irowb, semM)
        pltpu.async_copy(col_hbm.at[pl.ds(off, IC)], icolb, semM)
        pltpu.async_copy(val_hbm.at[pl.ds(off, IC)], ivalb, semM)

    def meta_drain(bufs):
        irowb, icolb, ivalb, semM = bufs
        pltpu.make_async_copy(row_hbm.at[pl.ds(0, IC)], irowb, semM).wait()
        pltpu.make_async_copy(col_hbm.at[pl.ds(0, IC)], icolb, semM).wait()
        pltpu.make_async_copy(val_hbm.at[pl.ds(0, IC)], ivalb, semM).wait()

    def process(carry, bufs, next_j):
        (fc0, off0, w0lo, w0hi, v0lo, v0hi,
         fc1, off1, w1lo, w1hi, v1lo, v1hi) = carry
        irowb, icolb, ivalb = bufs[0], bufs[1], bufs[2]
        meta_drain(bufs)
        for g in range(IC // 16):
            r = irowb[pl.ds(g * 16, 16)]
            cv = icolb[pl.ds(g * 16, 16)]
            vv = ivalb[pl.ds(g * 16, 16)]
            lr = jnp.where(r >= HALF, r - HALF, r)
            w0v = lax.shift_left(lr, 14) + cv
            for l in range(16):
                w0s = w0v[l]
                vs = vv[l]
                loc = r[l] < HALF
                t0 = jnp.where(loc, fc0, -1)
                t1 = jnp.where(loc, -1, fc1)
                m0lo = lane == t0
                m0hi = lane == (t0 - 16)
                m1lo = lane == t1
                m1hi = lane == (t1 - 16)
                w0lo = jnp.where(m0lo, w0s, w0lo)
                w0hi = jnp.where(m0hi, w0s, w0hi)
                v0lo = jnp.where(m0lo, vs, v0lo)
                v0hi = jnp.where(m0hi, vs, v0hi)
                w1lo = jnp.where(m1lo, w0s, w1lo)
                w1hi = jnp.where(m1hi, w0s, w1hi)
                v1lo = jnp.where(m1lo, vs, v1lo)
                v1hi = jnp.where(m1hi, vs, v1hi)
                inc = jnp.where(loc, 1, 0)
                fc0 = fc0 + inc
                fc1 = fc1 + (1 - inc)
            fl0 = fc0 >= 16

            @pl.when(fl0)
            def _(w0lo=w0lo, v0lo=v0lo, off0=off0):
                sw0[0][pl.ds(off0, 16)] = w0lo
                sv[0][pl.ds(off0, 16)] = v0lo

            w0lo = jnp.where(fl0, w0hi, w0lo)
            v0lo = jnp.where(fl0, v0hi, v0lo)
            fc0 = jnp.where(fl0, fc0 - 16, fc0)
            off0 = jnp.where(fl0, off0 + 16, off0)
            fl1 = fc1 >= 16

            @pl.when(fl1)
            def _(w1lo=w1lo, v1lo=v1lo, off1=off1):
                sw0[1][pl.ds(off1, 16)] = w1lo
                sv[1][pl.ds(off1, 16)] = v1lo

            w1lo = jnp.where(fl1, w1hi, w1lo)
            v1lo = jnp.where(fl1, v1hi, v1lo)
            fc1 = jnp.where(fl1, fc1 - 16, fc1)
            off1 = jnp.where(fl1, off1 + 16, off1)
        if next_j is not None:
            meta_issue(next_j, bufs)
        return (fc0, off0, w0lo, w0hi, v0lo, v0hi,
                fc1, off1, w1lo, w1hi, v1lo, v1hi)

    A = (irow, icol, ival, semA)
    Bb = (irow2, icol2, ival2, semB)
    NCH_IN = EPB // IC

    def pair(k, carry):
        carry = process(carry, A, 2 * k + 2)
        carry = process(carry, Bb, 2 * k + 3)
        return carry

    meta_issue(0, A)
    meta_issue(1, Bb)
    init = (jnp.int32(0), jnp.int32(0), zi, zi, zf, zf,
            jnp.int32(0), jnp.int32(0), zi, zi, zf, zf)
    carry = lax.fori_loop(0, NCH_IN // 2, pair, init)
    carry = process(carry, A, None)
    meta_drain(Bb)
    (fc0, off0, w0lo, _, v0lo, _,
     fc1, off1, w1lo, _, v1lo, _) = carry

    trash_w0 = jnp.broadcast_to((HALF + (w % NS)) << 14, (16,)).astype(jnp.int32)
    for h, fc, off, wlo, vlo in ((0, fc0, off0, w0lo, v0lo),
                                 (1, fc1, off1, w1lo, v1lo)):
        sw0[h][pl.ds(off, 16)] = jnp.where(lane < fc, wlo, trash_w0)
        sv[h][pl.ds(off, 16)] = jnp.where(lane < fc, vlo, zf)
        sw0[h][pl.ds(off + 16, 16)] = trash_w0
        sv[h][pl.ds(off + 16, 16)] = zf
        sw0[h][pl.ds(off + 32, 16)] = trash_w0
        sv[h][pl.ds(off + 32, 16)] = zf
        base = h * HSTRIDE + w * RCAP
        pltpu.sync_copy(sw0[h], bw0_out.at[pl.ds(base, RCAP)])
        pltpu.sync_copy(sv[h], bval_out.at[pl.ds(base, RCAP)])
        nch = (off + fc + (EC - 1)) // EC
        cbuf[pl.ds(0, 16)] = jnp.where(lane == 0, nch, 0)
        pltpu.sync_copy(cbuf, counts_out.at[pl.ds((h * NW + w) * 16, 16)])


_bucket = pl.kernel(
    _bucket_body,
    out_type=(jax.ShapeDtypeStruct((BLEN,), jnp.int32),
              jax.ShapeDtypeStruct((BLEN,), jnp.float32),
              jax.ShapeDtypeStruct((NC * NW * 16,), jnp.int32)),
    mesh=plsc.VectorSubcoreMesh(core_axis_name="c", subcore_axis_name="s"),
    scratch_types=[
        pltpu.VMEM((IC,), jnp.int32),
        pltpu.VMEM((IC,), jnp.int32),
        pltpu.VMEM((IC,), jnp.float32),
        pltpu.VMEM((IC,), jnp.int32),
        pltpu.VMEM((IC,), jnp.int32),
        pltpu.VMEM((IC,), jnp.float32),
        pltpu.VMEM((RCAP,), jnp.int32),
        pltpu.VMEM((RCAP,), jnp.float32),
        pltpu.VMEM((RCAP,), jnp.int32),
        pltpu.VMEM((RCAP,), jnp.float32),
        pltpu.VMEM((16,), jnp.int32),
        pltpu.SemaphoreType.DMA,
        pltpu.SemaphoreType.DMA,
    ],
)


def _hop_body(x_hbm, bw0, bval, counts, zero_hbm, out_hbm,
              x_sh, acc_sh, cbuf, tridx,
              lrowA, colA, rowsA, lrowB, colB, rowsB,
              mw0_0, mval_0, mw0_1, mval_1,
              semW0, semW1, semGA, semGB, semSA, semSB):
    c = lax.axis_index("c")
    s = lax.axis_index("s")
    lo = c * HALF
    trash = HALF + s

    pltpu.sync_copy(x_hbm.at[pl.ds(s * XROWS_PER_SUB, XROWS_PER_SUB)],
                    x_sh.at[pl.ds(s * XROWS_PER_SUB, XROWS_PER_SUB)])

    @pl.when(s == 0)
    def _():
        pltpu.sync_copy(x_hbm.at[pl.ds(NS * XROWS_PER_SUB, XTAIL)],
                        x_sh.at[pl.ds(NS * XROWS_PER_SUB, XTAIL)])

    @pl.when(s == 1)
    def _():
        pltpu.sync_copy(zero_hbm, acc_sh)

    for t in range(EC // 16):
        tridx[pl.ds(t * 16, 16)] = jnp.broadcast_to(trash, (16,)).astype(jnp.int32)
    plsc.subcore_barrier()

    A = (lrowA, colA, rowsA, semGA, semSA)
    Bb = (lrowB, colB, rowsB, semGB, semSB)
    W0w = (mw0_0, mval_0, semW0)
    W1w = (mw0_1, mval_1, semW1)
    WLEN = 4 * EC

    def win_issue(rbase, wi, wbufs):
        mw0, mval, semW = wbufs
        off = rbase + wi * WLEN
        pltpu.async_copy(bw0.at[pl.ds(off, WLEN)], mw0, semW)
        pltpu.async_copy(bval.at[pl.ds(off, WLEN)], mval, semW)

    def win_drain(rbase, wbufs):
        mw0, mval, semW = wbufs
        pltpu.make_async_copy(bw0.at[pl.ds(rbase, WLEN)], mw0, semW).wait()
        pltpu.make_async_copy(bval.at[pl.ds(rbase, WLEN)], mval, semW).wait()

    def start_phase(bufs, mw0ref, mbase):
        lrowb, colb, rowsb, semG, semS = bufs
        pltpu.make_async_copy(rowsb, acc_sh.at[lrowb], semS).wait()
        for g in range(EC // 16):
            wv = mw0ref[pl.ds(mbase + g * 16, 16)]
            lrowb[pl.ds(g * 16, 16)] = lax.shift_right_logical(wv, 14)
            colb[pl.ds(g * 16, 16)] = wv & 16383
        pltpu.async_copy(x_sh.at[colb], rowsb, semG)

    def finish_phase(bufs, mvalref, mbase):
        lrowb, colb, rowsb, semG, semS = bufs
        pltpu.make_async_copy(x_sh.at[colb], rowsb, semG).wait()

        def scale16(g, _):
            vv = mvalref[pl.ds(mbase + g * 16, 16)]
            for l in range(16):
                e = g * 16 + l
                v = vv[l]
                for q in range(D // 16):
                    rowsb[e, pl.ds(q * 16, 16)] = rowsb[e, pl.ds(q * 16, 16)] * v
            return 0

        lax.fori_loop(0, EC // 16, scale16, 0)
        pltpu.async_copy(rowsb, acc_sh.at[lrowb], semS)

    def quad(rbase, wbufs):
        mw0, mval = wbufs[0], wbufs[1]
        start_phase(A, mw0, 0)
        start_phase(Bb, mw0, EC)
        finish_phase(A, mval, 0)
        finish_phase(Bb, mval, EC)
        start_phase(A, mw0, 2 * EC)
        start_phase(Bb, mw0, 3 * EC)
        finish_phase(A, mval, 2 * EC)
        finish_phase(Bb, mval, 3 * EC)

    def do_region(ridx, nch):
        rbase = c * HSTRIDE + ridx * RCAP
        pltpu.async_copy(rowsA, acc_sh.at[tridx], semSA)
        pltpu.async_copy(rowsB, acc_sh.at[tridx], semSB)
        win_issue(rbase, 0, W0w)
        win_issue(rbase, 1, W1w)
        m8 = nch // 8
        rem = nch - m8 * 8

        def body8(m, carry):
            win_drain(rbase, W0w)
            quad(rbase, W0w)
            win_issue(rbase, 2 * m + 2, W0w)
            win_drain(rbase, W1w)
            quad(rbase, W1w)
            win_issue(rbase, 2 * m + 3, W1w)
            return carry

        lax.fori_loop(0, m8, body8, 0)

        def tail_chunk(i, wbufs):
            bufs = A if i % 2 == 0 else Bb
            mb = (i % 4) * EC

            @pl.when(rem > i)
            def _():
                start_phase(bufs, wbufs[0], mb)
                finish_phase(bufs, wbufs[1], mb)

        win_drain(rbase, W0w)
        for i in range(4):
            tail_chunk(i, W0w)
        win_drain(rbase, W1w)
        for i in range(4, 7):
            tail_chunk(i, W1w)

        pltpu.make_async_copy(rowsA, acc_sh.at[lrowA], semSA).wait()
        pltpu.make_async_copy(rowsB, acc_sh.at[lrowB], semSB).wait()

    pltpu.sync_copy(counts.at[pl.ds((c * NW + 2 * s) * 16, 32)], cbuf)
    n0 = cbuf[pl.ds(0, 16)][0]
    n1 = cbuf[pl.ds(16, 16)][0]
    do_region(2 * s, n0)
    do_region(2 * s + 1, n1)
    plsc.subcore_barrier()

    @pl.when(s == 0)
    def _():
        pltpu.sync_copy(acc_sh.at[pl.ds(0, HALF)], out_hbm.at[pl.ds(lo, HALF)])


_hop = pl.kernel(
    _hop_body,
    out_type=jax.ShapeDtypeStruct((NN, D), jnp.float32),
    mesh=plsc.VectorSubcoreMesh(core_axis_name="c", subcore_axis_name="s"),
    scratch_types=[
        pltpu.VMEM_SHARED((NN, D), jnp.float32),
        pltpu.VMEM_SHARED((ACC_ROWS, D), jnp.float32),
        pltpu.VMEM((32,), jnp.int32),
        pltpu.VMEM((EC,), jnp.int32),
        pltpu.VMEM((EC,), jnp.int32),
        pltpu.VMEM((EC,), jnp.int32),
        pltpu.VMEM((EC, D), jnp.float32),
        pltpu.VMEM((EC,), jnp.int32),
        pltpu.VMEM((EC,), jnp.int32),
        pltpu.VMEM((EC, D), jnp.float32),
        pltpu.VMEM((4 * EC,), jnp.int32),
        pltpu.VMEM((4 * EC,), jnp.float32),
        pltpu.VMEM((4 * EC,), jnp.int32),
        pltpu.VMEM((4 * EC,), jnp.float32),
        pltpu.SemaphoreType.DMA,
        pltpu.SemaphoreType.DMA,
        pltpu.SemaphoreType.DMA,
        pltpu.SemaphoreType.DMA,
        pltpu.SemaphoreType.DMA,
        pltpu.SemaphoreType.DMA,
    ],
)


BPW = BATCH // NW


def _loss_body(x0, x1, x2, x3, u_hbm, i_hbm, n_hbm, pos_out, neg_out,
               uidx, iidx, nidx, bufs_flat, pos_sm, neg_sm, sem):
    c = lax.axis_index("c")
    s = lax.axis_index("s")
    w = s * NC + c
    b0 = w * BPW
    tables = (x0, x1, x2, x3)
    dsts = bufs_flat[0:6]
    tsets = (bufs_flat[6:12], bufs_flat[12:18], bufs_flat[18:24])
    usum, isum, ng0, ng1, ng2, ng3 = dsts
    ngs = (ng0, ng1, ng2, ng3)

    pltpu.async_copy(u_hbm.at[pl.ds(b0, BPW)], uidx, sem)
    pltpu.async_copy(i_hbm.at[pl.ds(b0, BPW)], iidx, sem)
    for n in range(NNEG):
        pltpu.async_copy(n_hbm.at[pl.ds(n * BATCH + b0, BPW)],
                         nidx.at[pl.ds(n * BPW, BPW)], sem)
    pltpu.make_async_copy(u_hbm.at[pl.ds(b0, BPW)], uidx, sem).wait()
    pltpu.make_async_copy(i_hbm.at[pl.ds(b0, BPW)], iidx, sem).wait()
    for n in range(NNEG):
        pltpu.make_async_copy(n_hbm.at[pl.ds(n * BATCH + b0, BPW)],
                              nidx.at[pl.ds(n * BPW, BPW)], sem).wait()

    idxs = (uidx.at[pl.ds(0, BPW)], iidx.at[pl.ds(0, BPW)],
            nidx.at[pl.ds(0, BPW)], nidx.at[pl.ds(BPW, BPW)],
            nidx.at[pl.ds(2 * BPW, BPW)], nidx.at[pl.ds(3 * BPW, BPW)])
    for r in range(6):
        pltpu.async_copy(tables[0].at[idxs[r]], dsts[r], sem)
    for t in range(1, 4):
        for r in range(6):
            pltpu.async_copy(tables[t].at[idxs[r]], tsets[t - 1][r], sem)
    for r in range(6):
        pltpu.make_async_copy(tables[0].at[idxs[r]], dsts[r], sem).wait()
    for t in range(1, 4):
        for r in range(6):
            pltpu.make_async_copy(tables[t].at[idxs[r]], tsets[t - 1][r],
                                  sem).wait()

    for t in range(1, 4):
        for r in range(6):

            def addloop(i, _, t=t, r=r):
                for q in range(D // 16):
                    dsts[r][i, pl.ds(q * 16, 16)] = (
                        dsts[r][i, pl.ds(q * 16, 16)]
                        + tsets[t - 1][r][i, pl.ds(q * 16, 16)])
                return 0

            lax.fori_loop(0, BPW, addloop, 0)

    def dots(b, _):
        pacc = jnp.zeros((16,), jnp.float32)
        for q in range(D // 16):
            pacc = pacc + (usum[b, pl.ds(q * 16, 16)]
                           * isum[b, pl.ds(q * 16, 16)])
        pos_sm[b, pl.ds(0, 16)] = pacc
        for n in range(NNEG):
            nacc = jnp.zeros((16,), jnp.float32)
            for q in range(D // 16):
                nacc = nacc + (usum[b, pl.ds(q * 16, 16)]
                               * ngs[n][b, pl.ds(q * 16, 16)])
            neg_sm[n * BPW + b, pl.ds(0, 16)] = nacc
        return 0

    lax.fori_loop(0, BPW, dots, 0)
    pltpu.sync_copy(pos_sm, pos_out.at[pl.ds(b0, BPW)])
    for n in range(NNEG):
        pltpu.sync_copy(neg_sm.at[pl.ds(n * BPW, BPW)],
                        neg_out.at[pl.ds(n * BATCH + b0, BPW)])


_loss = pl.kernel(
    _loss_body,
    out_type=(jax.ShapeDtypeStruct((BATCH, 16), jnp.float32),
              jax.ShapeDtypeStruct((NNEG * BATCH, 16), jnp.float32)),
    mesh=plsc.VectorSubcoreMesh(core_axis_name="c", subcore_axis_name="s"),
    scratch_types=[
        pltpu.VMEM((BPW,), jnp.int32),
        pltpu.VMEM((BPW,), jnp.int32),
        pltpu.VMEM((NNEG * BPW,), jnp.int32),
        [pltpu.VMEM((BPW, D), jnp.float32) for _ in range(24)],
        pltpu.VMEM((BPW, 16), jnp.float32),
        pltpu.VMEM((NNEG * BPW, 16), jnp.float32),
        pltpu.SemaphoreType.DMA,
    ],
)


def _nce_body(p_ref, n_ref, o_ref):
    p = jnp.sum(p_ref[...], axis=-1) * (1.0 / 16.0)
    nk = jnp.sum(n_ref[...], axis=-1) * (1.0 / 16.0)
    ne = jnp.sum(jnp.exp(nk), axis=0)
    loss = jnp.mean(jnp.log(jnp.exp(p) + ne) - p)
    o_ref[...] = jnp.reshape(loss, (1, 1))


_nce = pl.pallas_call(
    _nce_body,
    out_shape=jax.ShapeDtypeStruct((1, 1), jnp.float32),
)


def kernel(edge_vals, user_emb, item_emb, users, items, negatives, edge_index):
    all_emb = jnp.concatenate([user_emb, item_emb], axis=0).astype(jnp.float32)
    padi = jnp.zeros((IC,), jnp.int32)
    row = jnp.concatenate([edge_index[0].astype(jnp.int32), padi])
    col = jnp.concatenate([edge_index[1].astype(jnp.int32), padi])
    ev = jnp.concatenate([edge_vals.astype(jnp.float32),
                          jnp.zeros((IC,), jnp.float32)])
    zero_acc = jnp.zeros((ACC_ROWS, D), jnp.float32)

    bw0, bval, counts = _bucket(row, col, ev)

    x0 = all_emb
    x1 = _hop(x0, bw0, bval, counts, zero_acc)
    x2 = _hop(x1, bw0, bval, counts, zero_acc)
    x3 = _hop(x2, bw0, bval, counts, zero_acc)

    u = users.astype(jnp.int32)
    it = items.astype(jnp.int32) + N_USERS
    ng = negatives.astype(jnp.int32) + N_USERS
    pos, negk = _loss(x0, x1, x2, x3, u, it, ng)
    out = _nce(pos, negk.reshape(NNEG, BATCH, 16))
    return out[0, 0]


# speedup vs baseline: 1.9944x; 1.0003x over previous
import jax
import jax.numpy as jnp
from jax import lax
from jax.experimental import pallas as pl
from jax.experimental.pallas import tpu as pltpu
from jax.experimental.pallas import tpu_sc as plsc

N_USERS = 2000
N_ITEMS = 8000
NN = N_USERS + N_ITEMS
D = 128
HOPS = 3
NNEG = 4
BATCH = 1024
E = 320000

NC = 2
NS = 16
NW = NC * NS
HALF = NN // NC
ACC_ROWS = 5024
EC = 32

EPB = E // NW
IC = 80
RCAP = 10048
RCH = RCAP // EC
HSTRIDE = NW * RCAP
BLEN = NC * HSTRIDE + 8 * EC

XROWS_PER_SUB = 624
XTAIL = NN - NS * XROWS_PER_SUB


def _bucket_body(row_hbm, col_hbm, val_hbm,
                 bw0_out, bval_out, counts_out,
                 irow, icol, ival, irow2, icol2, ival2,
                 sw0_0, sv_0, sw0_1, sv_1, cbuf, semA, semB):
    c = lax.axis_index("c")
    s = lax.axis_index("s")
    w = s * NC + c
    ebase = w * EPB
    lane = lax.iota(jnp.int32, 16)
    sw0 = (sw0_0, sw0_1)
    sv = (sv_0, sv_1)
    zi = jnp.zeros((16,), jnp.int32)
    zf = jnp.zeros((16,), jnp.float32)

    def meta_issue(j, bufs):
        irowb, icolb, ivalb, semM = bufs
        off = ebase + j * IC
        pltpu.async_copy(row_hbm.at[pl.ds(off, IC)], irowb, semM)
        pltpu.async_copy(col_hbm.at[pl.ds(off, IC)], icolb, semM)
        pltpu.async_copy(val_hbm.at[pl.ds(off, IC)], ivalb, semM)

    def meta_drain(bufs):
        irowb, icolb, ivalb, semM = bufs
        pltpu.make_async_copy(row_hbm.at[pl.ds(0, IC)], irowb, semM).wait()
        pltpu.make_async_copy(col_hbm.at[pl.ds(0, IC)], icolb, semM).wait()
        pltpu.make_async_copy(val_hbm.at[pl.ds(0, IC)], ivalb, semM).wait()

    def process(carry, bufs, next_j):
        (fc0, off0, w0lo, w0hi, v0lo, v0hi,
         fc1, off1, w1lo, w1hi, v1lo, v1hi) = carry
        irowb, icolb, ivalb = bufs[0], bufs[1], bufs[2]
        meta_drain(bufs)
        for g in range(IC // 16):
            r = irowb[pl.ds(g * 16, 16)]
            cv = icolb[pl.ds(g * 16, 16)]
            vv = ivalb[pl.ds(g * 16, 16)]
            lr = jnp.where(r >= HALF, r - HALF, r)
            w0v = lax.shift_left(lr, 14) + cv
            for l in range(16):
                w0s = w0v[l]
                vs = vv[l]
                loc = r[l] < HALF
                t0 = jnp.where(loc, fc0, -1)
                t1 = jnp.where(loc, -1, fc1)
                m0lo = lane == t0
                m0hi = lane == (t0 - 16)
                m1lo = lane == t1
                m1hi = lane == (t1 - 16)
                w0lo = jnp.where(m0lo, w0s, w0lo)
                w0hi = jnp.where(m0hi, w0s, w0hi)
                v0lo = jnp.where(m0lo, vs, v0lo)
                v0hi = jnp.where(m0hi, vs, v0hi)
                w1lo = jnp.where(m1lo, w0s, w1lo)
                w1hi = jnp.where(m1hi, w0s, w1hi)
                v1lo = jnp.where(m1lo, vs, v1lo)
                v1hi = jnp.where(m1hi, vs, v1hi)
                inc = jnp.where(loc, 1, 0)
                fc0 = fc0 + inc
                fc1 = fc1 + (1 - inc)
            fl0 = fc0 >= 16

            @pl.when(fl0)
            def _(w0lo=w0lo, v0lo=v0lo, off0=off0):
                sw0[0][pl.ds(off0, 16)] = w0lo
                sv[0][pl.ds(off0, 16)] = v0lo

            w0lo = jnp.where(fl0, w0hi, w0lo)
            v0lo = jnp.where(fl0, v0hi, v0lo)
            fc0 = jnp.where(fl0, fc0 - 16, fc0)
            off0 = jnp.where(fl0, off0 + 16, off0)
            fl1 = fc1 >= 16

            @pl.when(fl1)
            def _(w1lo=w1lo, v1lo=v1lo, off1=off1):
                sw0[1][pl.ds(off1, 16)] = w1lo
                sv[1][pl.ds(off1, 16)] = v1lo

            w1lo = jnp.where(fl1, w1hi, w1lo)
            v1lo = jnp.where(fl1, v1hi, v1lo)
            fc1 = jnp.where(fl1, fc1 - 16, fc1)
            off1 = jnp.where(fl1, off1 + 16, off1)
        if next_j is not None:
            meta_issue(next_j, bufs)
        return (fc0, off0, w0lo, w0hi, v0lo, v0hi,
                fc1, off1, w1lo, w1hi, v1lo, v1hi)

    A = (irow, icol, ival, semA)
    Bb = (irow2, icol2, ival2, semB)
    NCH_IN = EPB // IC

    def pair(k, carry):
        carry = process(carry, A, 2 * k + 2)
        carry = process(carry, Bb, 2 * k + 3)
        return carry

    meta_issue(0, A)
    meta_issue(1, Bb)
    init = (jnp.int32(0), jnp.int32(0), zi, zi, zf, zf,
            jnp.int32(0), jnp.int32(0), zi, zi, zf, zf)
    carry = lax.fori_loop(0, NCH_IN // 2, pair, init)
    carry = process(carry, A, None)
    meta_drain(Bb)
    (fc0, off0, w0lo, _, v0lo, _,
     fc1, off1, w1lo, _, v1lo, _) = carry

    trash_w0 = jnp.broadcast_to((HALF + (w % NS)) << 14, (16,)).astype(jnp.int32)
    for h, fc, off, wlo, vlo in ((0, fc0, off0, w0lo, v0lo),
                                 (1, fc1, off1, w1lo, v1lo)):
        sw0[h][pl.ds(off, 16)] = jnp.where(lane < fc, wlo, trash_w0)
        sv[h][pl.ds(off, 16)] = jnp.where(lane < fc, vlo, zf)
        sw0[h][pl.ds(off + 16, 16)] = trash_w0
        sv[h][pl.ds(off + 16, 16)] = zf
        sw0[h][pl.ds(off + 32, 16)] = trash_w0
        sv[h][pl.ds(off + 32, 16)] = zf
        base = h * HSTRIDE + w * RCAP
        pltpu.sync_copy(sw0[h], bw0_out.at[pl.ds(base, RCAP)])
        pltpu.sync_copy(sv[h], bval_out.at[pl.ds(base, RCAP)])
        nch = (off + fc + (EC - 1)) // EC
        cbuf[pl.ds(0, 16)] = jnp.where(lane == 0, nch, 0)
        pltpu.sync_copy(cbuf, counts_out.at[pl.ds((h * NW + w) * 16, 16)])


_bucket = pl.kernel(
    _bucket_body,
    out_type=(jax.ShapeDtypeStruct((BLEN,), jnp.int32),
              jax.ShapeDtypeStruct((BLEN,), jnp.float32),
              jax.ShapeDtypeStruct((NC * NW * 16,), jnp.int32)),
    mesh=plsc.VectorSubcoreMesh(core_axis_name="c", subcore_axis_name="s"),
    scratch_types=[
        pltpu.VMEM((IC,), jnp.int32),
        pltpu.VMEM((IC,), jnp.int32),
        pltpu.VMEM((IC,), jnp.float32),
        pltpu.VMEM((IC,), jnp.int32),
        pltpu.VMEM((IC,), jnp.int32),
        pltpu.VMEM((IC,), jnp.float32),
        pltpu.VMEM((RCAP,), jnp.int32),
        pltpu.VMEM((RCAP,), jnp.float32),
        pltpu.VMEM((RCAP,), jnp.int32),
        pltpu.VMEM((RCAP,), jnp.float32),
        pltpu.VMEM((16,), jnp.int32),
        pltpu.SemaphoreType.DMA,
        pltpu.SemaphoreType.DMA,
    ],
)


def _hop_body(x_hbm, bw0, bval, counts, zero_hbm, out_hbm,
              x_sh, acc_sh, cbuf, tridx,
              lrowA, colA, rowsA, lrowB, colB, rowsB,
              mw0_0, mval_0, mw0_1, mval_1,
              semW0, semW1, semGA, semGB, semSA, semSB):
    c = lax.axis_index("c")
    s = lax.axis_index("s")
    lo = c * HALF
    trash = HALF + s

    pltpu.sync_copy(x_hbm.at[pl.ds(s * XROWS_PER_SUB, XROWS_PER_SUB)],
                    x_sh.at[pl.ds(s * XROWS_PER_SUB, XROWS_PER_SUB)])

    @pl.when(s == 0)
    def _():
        pltpu.sync_copy(x_hbm.at[pl.ds(NS * XROWS_PER_SUB, XTAIL)],
                        x_sh.at[pl.ds(NS * XROWS_PER_SUB, XTAIL)])

    @pl.when(s < 15)
    def _():
        pltpu.sync_copy(zero_hbm.at[pl.ds(s * 320, 320)],
                        acc_sh.at[pl.ds(s * 320, 320)])

    @pl.when(s == 15)
    def _():
        pltpu.sync_copy(zero_hbm.at[pl.ds(4800, 224)],
                        acc_sh.at[pl.ds(4800, 224)])

    for t in range(EC // 16):
        tridx[pl.ds(t * 16, 16)] = jnp.broadcast_to(trash, (16,)).astype(jnp.int32)
    plsc.subcore_barrier()

    A = (lrowA, colA, rowsA, semGA, semSA)
    Bb = (lrowB, colB, rowsB, semGB, semSB)
    W0w = (mw0_0, mval_0, semW0)
    W1w = (mw0_1, mval_1, semW1)
    WLEN = 4 * EC

    def win_issue(rbase, wi, wbufs):
        mw0, mval, semW = wbufs
        off = rbase + wi * WLEN
        pltpu.async_copy(bw0.at[pl.ds(off, WLEN)], mw0, semW)
        pltpu.async_copy(bval.at[pl.ds(off, WLEN)], mval, semW)

    def win_drain(rbase, wbufs):
        mw0, mval, semW = wbufs
        pltpu.make_async_copy(bw0.at[pl.ds(rbase, WLEN)], mw0, semW).wait()
        pltpu.make_async_copy(bval.at[pl.ds(rbase, WLEN)], mval, semW).wait()

    def start_phase(bufs, mw0ref, mbase):
        lrowb, colb, rowsb, semG, semS = bufs
        pltpu.make_async_copy(rowsb, acc_sh.at[lrowb], semS).wait()
        for g in range(EC // 16):
            wv = mw0ref[pl.ds(mbase + g * 16, 16)]
            lrowb[pl.ds(g * 16, 16)] = lax.shift_right_logical(wv, 14)
            colb[pl.ds(g * 16, 16)] = wv & 16383
        pltpu.async_copy(x_sh.at[colb], rowsb, semG)

    def finish_phase(bufs, mvalref, mbase):
        lrowb, colb, rowsb, semG, semS = bufs
        pltpu.make_async_copy(x_sh.at[colb], rowsb, semG).wait()

        def scale16(g, _):
            vv = mvalref[pl.ds(mbase + g * 16, 16)]
            for l in range(16):
                e = g * 16 + l
                v = vv[l]
                for q in range(D // 16):
                    rowsb[e, pl.ds(q * 16, 16)] = rowsb[e, pl.ds(q * 16, 16)] * v
            return 0

        lax.fori_loop(0, EC // 16, scale16, 0)
        pltpu.async_copy(rowsb, acc_sh.at[lrowb], semS)

    def quad(rbase, wbufs):
        mw0, mval = wbufs[0], wbufs[1]
        start_phase(A, mw0, 0)
        start_phase(Bb, mw0, EC)
        finish_phase(A, mval, 0)
        finish_phase(Bb, mval, EC)
        start_phase(A, mw0, 2 * EC)
        start_phase(Bb, mw0, 3 * EC)
        finish_phase(A, mval, 2 * EC)
        finish_phase(Bb, mval, 3 * EC)

    def do_region(ridx, nch):
        rbase = c * HSTRIDE + ridx * RCAP
        pltpu.async_copy(rowsA, acc_sh.at[tridx], semSA)
        pltpu.async_copy(rowsB, acc_sh.at[tridx], semSB)
        win_issue(rbase, 0, W0w)
        win_issue(rbase, 1, W1w)
        m8 = nch // 8
        rem = nch - m8 * 8

        def body8(m, carry):
            win_drain(rbase, W0w)
            quad(rbase, W0w)
            win_issue(rbase, 2 * m + 2, W0w)
            win_drain(rbase, W1w)
            quad(rbase, W1w)
            win_issue(rbase, 2 * m + 3, W1w)
            return carry

        lax.fori_loop(0, m8, body8, 0)

        def tail_chunk(i, wbufs):
            bufs = A if i % 2 == 0 else Bb
            mb = (i % 4) * EC

            @pl.when(rem > i)
            def _():
                start_phase(bufs, wbufs[0], mb)
                finish_phase(bufs, wbufs[1], mb)

        win_drain(rbase, W0w)
        for i in range(4):
            tail_chunk(i, W0w)
        win_drain(rbase, W1w)
        for i in range(4, 7):
            tail_chunk(i, W1w)

        pltpu.make_async_copy(rowsA, acc_sh.at[lrowA], semSA).wait()
        pltpu.make_async_copy(rowsB, acc_sh.at[lrowB], semSB).wait()

    pltpu.sync_copy(counts.at[pl.ds((c * NW + 2 * s) * 16, 32)], cbuf)
    n0 = cbuf[pl.ds(0, 16)][0]
    n1 = cbuf[pl.ds(16, 16)][0]
    do_region(2 * s, n0)
    do_region(2 * s + 1, n1)
    plsc.subcore_barrier()

    @pl.when(s < 15)
    def _():
        pltpu.sync_copy(acc_sh.at[pl.ds(s * 312, 312)],
                        out_hbm.at[pl.ds(lo + s * 312, 312)])

    @pl.when(s == 15)
    def _():
        pltpu.sync_copy(acc_sh.at[pl.ds(4680, 320)],
                        out_hbm.at[pl.ds(lo + 4680, 320)])


_hop = pl.kernel(
    _hop_body,
    out_type=jax.ShapeDtypeStruct((NN, D), jnp.float32),
    mesh=plsc.VectorSubcoreMesh(core_axis_name="c", subcore_axis_name="s"),
    scratch_types=[
        pltpu.VMEM_SHARED((NN, D), jnp.float32),
        pltpu.VMEM_SHARED((ACC_ROWS, D), jnp.float32),
        pltpu.VMEM((32,), jnp.int32),
        pltpu.VMEM((EC,), jnp.int32),
        pltpu.VMEM((EC,), jnp.int32),
        pltpu.VMEM((EC,), jnp.int32),
        pltpu.VMEM((EC, D), jnp.float32),
        pltpu.VMEM((EC,), jnp.int32),
        pltpu.VMEM((EC,), jnp.int32),
        pltpu.VMEM((EC, D), jnp.float32),
        pltpu.VMEM((4 * EC,), jnp.int32),
        pltpu.VMEM((4 * EC,), jnp.float32),
        pltpu.VMEM((4 * EC,), jnp.int32),
        pltpu.VMEM((4 * EC,), jnp.float32),
        pltpu.SemaphoreType.DMA,
        pltpu.SemaphoreType.DMA,
        pltpu.SemaphoreType.DMA,
        pltpu.SemaphoreType.DMA,
        pltpu.SemaphoreType.DMA,
        pltpu.SemaphoreType.DMA,
    ],
)


BPW = BATCH // NW


def _loss_body(x0, x1, x2, x3, u_hbm, i_hbm, n_hbm, pos_out, neg_out,
               uidx, iidx, nidx, bufs_flat, pos_sm, neg_sm, sem):
    c = lax.axis_index("c")
    s = lax.axis_index("s")
    w = s * NC + c
    b0 = w * BPW
    tables = (x0, x1, x2, x3)
    dsts = bufs_flat[0:6]
    tsets = (bufs_flat[6:12], bufs_flat[12:18], bufs_flat[18:24])
    usum, isum, ng0, ng1, ng2, ng3 = dsts
    ngs = (ng0, ng1, ng2, ng3)

    pltpu.async_copy(u_hbm.at[pl.ds(b0, BPW)], uidx, sem)
    pltpu.async_copy(i_hbm.at[pl.ds(b0, BPW)], iidx, sem)
    for n in range(NNEG):
        pltpu.async_copy(n_hbm.at[pl.ds(n * BATCH + b0, BPW)],
                         nidx.at[pl.ds(n * BPW, BPW)], sem)
    pltpu.make_async_copy(u_hbm.at[pl.ds(b0, BPW)], uidx, sem).wait()
    pltpu.make_async_copy(i_hbm.at[pl.ds(b0, BPW)], iidx, sem).wait()
    for n in range(NNEG):
        pltpu.make_async_copy(n_hbm.at[pl.ds(n * BATCH + b0, BPW)],
                              nidx.at[pl.ds(n * BPW, BPW)], sem).wait()

    idxs = (uidx.at[pl.ds(0, BPW)], iidx.at[pl.ds(0, BPW)],
            nidx.at[pl.ds(0, BPW)], nidx.at[pl.ds(BPW, BPW)],
            nidx.at[pl.ds(2 * BPW, BPW)], nidx.at[pl.ds(3 * BPW, BPW)])
    for r in range(6):
        pltpu.async_copy(tables[0].at[idxs[r]], dsts[r], sem)
    for t in range(1, 4):
        for r in range(6):
            pltpu.async_copy(tables[t].at[idxs[r]], tsets[t - 1][r], sem)
    for r in range(6):
        pltpu.make_async_copy(tables[0].at[idxs[r]], dsts[r], sem).wait()
    for t in range(1, 4):
        for r in range(6):
            pltpu.make_async_copy(tables[t].at[idxs[r]], tsets[t - 1][r],
                                  sem).wait()

    for t in range(1, 4):
        for r in range(6):

            def addloop(i, _, t=t, r=r):
                for q in range(D // 16):
                    dsts[r][i, pl.ds(q * 16, 16)] = (
                        dsts[r][i, pl.ds(q * 16, 16)]
                        + tsets[t - 1][r][i, pl.ds(q * 16, 16)])
                return 0

            lax.fori_loop(0, BPW, addloop, 0)

    def dots(b, _):
        pacc = jnp.zeros((16,), jnp.float32)
        for q in range(D // 16):
            pacc = pacc + (usum[b, pl.ds(q * 16, 16)]
                           * isum[b, pl.ds(q * 16, 16)])
        pos_sm[b, pl.ds(0, 16)] = pacc
        for n in range(NNEG):
            nacc = jnp.zeros((16,), jnp.float32)
            for q in range(D // 16):
                nacc = nacc + (usum[b, pl.ds(q * 16, 16)]
                               * ngs[n][b, pl.ds(q * 16, 16)])
            neg_sm[n * BPW + b, pl.ds(0, 16)] = nacc
        return 0

    lax.fori_loop(0, BPW, dots, 0)
    pltpu.sync_copy(pos_sm, pos_out.at[pl.ds(b0, BPW)])
    for n in range(NNEG):
        pltpu.sync_copy(neg_sm.at[pl.ds(n * BPW, BPW)],
                        neg_out.at[pl.ds(n * BATCH + b0, BPW)])


_loss = pl.kernel(
    _loss_body,
    out_type=(jax.ShapeDtypeStruct((BATCH, 16), jnp.float32),
              jax.ShapeDtypeStruct((NNEG * BATCH, 16), jnp.float32)),
    mesh=plsc.VectorSubcoreMesh(core_axis_name="c", subcore_axis_name="s"),
    scratch_types=[
        pltpu.VMEM((BPW,), jnp.int32),
        pltpu.VMEM((BPW,), jnp.int32),
        pltpu.VMEM((NNEG * BPW,), jnp.int32),
        [pltpu.VMEM((BPW, D), jnp.float32) for _ in range(24)],
        pltpu.VMEM((BPW, 16), jnp.float32),
        pltpu.VMEM((NNEG * BPW, 16), jnp.float32),
        pltpu.SemaphoreType.DMA,
    ],
)


def _nce_body(p_ref, n_ref, o_ref):
    p = jnp.sum(p_ref[...], axis=-1) * (1.0 / 16.0)
    nk = jnp.sum(n_ref[...], axis=-1) * (1.0 / 16.0)
    ne = jnp.sum(jnp.exp(nk), axis=0)
    loss = jnp.mean(jnp.log(jnp.exp(p) + ne) - p)
    o_ref[...] = jnp.reshape(loss, (1, 1))


_nce = pl.pallas_call(
    _nce_body,
    out_shape=jax.ShapeDtypeStruct((1, 1), jnp.float32),
)


def kernel(edge_vals, user_emb, item_emb, users, items, negatives, edge_index):
    all_emb = jnp.concatenate([user_emb, item_emb], axis=0).astype(jnp.float32)
    padi = jnp.zeros((IC,), jnp.int32)
    row = jnp.concatenate([edge_index[0].astype(jnp.int32), padi])
    col = jnp.concatenate([edge_index[1].astype(jnp.int32), padi])
    ev = jnp.concatenate([edge_vals.astype(jnp.float32),
                          jnp.zeros((IC,), jnp.float32)])
    zero_acc = jnp.zeros((ACC_ROWS, D), jnp.float32)

    bw0, bval, counts = _bucket(row, col, ev)

    x0 = all_emb
    x1 = _hop(x0, bw0, bval, counts, zero_acc)
    x2 = _hop(x1, bw0, bval, counts, zero_acc)
    x3 = _hop(x2, bw0, bval, counts, zero_acc)

    u = users.astype(jnp.int32)
    it = items.astype(jnp.int32) + N_USERS
    ng = negatives.astype(jnp.int32) + N_USERS
    pos, negk = _loss(x0, x1, x2, x3, u, it, ng)
    out = _nce(pos, negk.reshape(NNEG, BATCH, 16))
    return out[0, 0]
